# Initial kernel scaffold; baseline (speedup 1.0000x reference)
import jax
import jax.numpy as jnp
from jax.experimental import pallas as pl


def kernel(x, edge_index, batch, W1, b1, W2, b2, W3, b3, fW1, fb1, fW2, fb2):
    raise NotImplementedError("write your pallas kernel here")


# speedup vs baseline: 10.4494x; 10.4494x over previous
import functools
import jax
import jax.numpy as jnp
from jax import lax
from jax.experimental import pallas as pl
from jax.experimental.pallas import tpu as pltpu
from jax.experimental.pallas import tpu_sc as plsc

N = 50000
E = 800000
G = 2000
NPAD = 51200
NROWS = 50176
GPAD = 2048
ECH = E // 128
BCH = NROWS // 128
NEG_INF = float("-inf")

_MESH = plsc.VectorSubcoreMesh(core_axis_name="c", subcore_axis_name="s")
_Z16 = functools.partial(jnp.zeros, (16,), jnp.float32)


def _fill(ref, n, value):
    v = jnp.full((16,), value, jnp.float32)

    def body(i, _):
        ref[pl.ds(i * 16, 16)] = v
        return 0

    lax.fori_loop(0, n // 16, body, 0)


def _fill2d(ref, rows, value):
    v = jnp.full((16,), value, jnp.float32)

    def body(i, _):
        r = i // 2
        col = (i % 2) * 16
        ref[r, pl.ds(col, 16)] = v
        return 0

    lax.fori_loop(0, rows * 2, body, 0)


def _sc_deg_counts(dst_hbm, bat_hbm, degp_hbm, cntp_hbm,
                   accd, accc, zbuf, obuf, idx):
    c = lax.axis_index("c")
    s = lax.axis_index("s")
    w = c * 16 + s
    _fill(zbuf, 3200, 0.0)
    _fill(obuf, 128, 1.0)
    pltpu.sync_copy(zbuf, accd.at[pl.ds(s * 3200, 3200)])
    pltpu.sync_copy(zbuf.at[pl.ds(0, 128)], accc.at[pl.ds(s * 128, 128)])
    plsc.subcore_barrier()

    def estep(k, _):
        pltpu.sync_copy(dst_hbm.at[pl.ds(k * 128, 128)], idx.at[0])
        pltpu.sync_copy(obuf, accd.at[idx.at[0]], add=True)
        return 0

    lax.fori_loop(w * ECH // 32, (w + 1) * ECH // 32, estep, 0)

    def bstep(k, _):
        pltpu.sync_copy(bat_hbm.at[pl.ds(k * 128, 128)], idx.at[0])
        pltpu.sync_copy(obuf, accc.at[idx.at[0]], add=True)
        return 0

    lax.fori_loop(w * BCH // 32, (w + 1) * BCH // 32, bstep, 0)

    plsc.subcore_barrier()
    pltpu.sync_copy(accd.at[pl.ds(s * 3200, 3200)],
                    degp_hbm.at[c].at[pl.ds(s * 3200, 3200)])
    pltpu.sync_copy(accc.at[pl.ds(s * 128, 128)],
                    cntp_hbm.at[c].at[pl.ds(s * 128, 128)])


_deg_counts = pl.kernel(
    _sc_deg_counts,
    out_type=[jax.ShapeDtypeStruct((2, NPAD), jnp.float32),
              jax.ShapeDtypeStruct((2, GPAD), jnp.float32)],
    mesh=_MESH,
    scratch_types=[
        pltpu.VMEM_SHARED((NPAD,), jnp.float32),
        pltpu.VMEM_SHARED((GPAD,), jnp.float32),
        pltpu.VMEM((3200,), jnp.float32),
        pltpu.VMEM((128,), jnp.float32),
        pltpu.VMEM((1, 128), jnp.int32),
    ],
)


def _sc_p0(xt_hbm, src_hbm, dst_hbm, p0p_hbm,
           acc, zbuf, vals, idxs, idxd, sem):
    c = lax.axis_index("c")
    s = lax.axis_index("s")
    w = c * 16 + s
    _fill(zbuf, 3200, 0.0)
    pltpu.sync_copy(zbuf, acc.at[pl.ds(s * 3200, 3200)])
    plsc.subcore_barrier()

    def estep(k, _):
        b = k * 128
        pltpu.sync_copy(src_hbm.at[pl.ds(b, 128)], idxs.at[0])
        pltpu.async_copy(xt_hbm.at[idxs.at[0]], vals, sem).wait()
        pltpu.sync_copy(dst_hbm.at[pl.ds(b, 128)], idxd.at[0])
        pltpu.sync_copy(vals, acc.at[idxd.at[0]], add=True)
        return 0

    lax.fori_loop(w * ECH // 32, (w + 1) * ECH // 32, estep, 0)

    plsc.subcore_barrier()
    pltpu.sync_copy(acc.at[pl.ds(s * 3200, 3200)],
                    p0p_hbm.at[c].at[pl.ds(s * 3200, 3200)])


_p0_scatter = pl.kernel(
    _sc_p0,
    out_type=[jax.ShapeDtypeStruct((2, NPAD), jnp.float32)],
    mesh=_MESH,
    scratch_types=[
        pltpu.VMEM_SHARED((NPAD,), jnp.float32),
        pltpu.VMEM((3200,), jnp.float32),
        pltpu.VMEM((128,), jnp.float32),
        pltpu.VMEM((1, 128), jnp.int32),
        pltpu.VMEM((1, 128), jnp.int32),
        pltpu.SemaphoreType.DMA,
    ],
)


def _make_row_scatter(num_chunks):
    per_core = num_chunks // 2

    def body(ht_hbm, src_hbm, dst_hbm, out_hbm,
             acc, zrows, rows, idxs, idxd, sem):
        c = lax.axis_index("c")
        s = lax.axis_index("s")
        _fill2d(zrows, 400, 0.0)

        for j in range(per_core):
            ch = c * per_core + j

            def zstep(i, _):
                pltpu.sync_copy(zrows, acc.at[pl.ds(s * 3200 + i * 400, 400), :])
                return 0

            lax.fori_loop(0, 8, zstep, 0)
            plsc.subcore_barrier()

            def estep(k, _):
                b = k * 128
                pltpu.sync_copy(src_hbm.at[pl.ds(b, 128)], idxs.at[0])
                pltpu.async_copy(ht_hbm.at[ch].at[idxs.at[0]], rows, sem).wait()
                pltpu.sync_copy(dst_hbm.at[pl.ds(b, 128)], idxd.at[0])
                pltpu.sync_copy(rows, acc.at[idxd.at[0]], add=True)
                return 0

            lax.fori_loop(s * ECH // 16, (s + 1) * ECH // 16, estep, 0)

            plsc.subcore_barrier()
            pltpu.sync_copy(acc.at[pl.ds(s * 3200, 3200), :],
                            out_hbm.at[ch].at[pl.ds(s * 3200, 3200), :])
            plsc.subcore_barrier()

    return pl.kernel(
        body,
        out_type=[jax.ShapeDtypeStruct((num_chunks, NPAD, 32), jnp.float32)],
        mesh=_MESH,
        compiler_params=pltpu.CompilerParams(use_tc_tiling_on_sc=False),
        scratch_types=[
            pltpu.VMEM_SHARED((NPAD, 32), jnp.float32),
            pltpu.VMEM((400, 32), jnp.float32),
            pltpu.VMEM((128, 32), jnp.float32),
            pltpu.VMEM((1, 128), jnp.int32),
            pltpu.VMEM((1, 128), jnp.int32),
            pltpu.SemaphoreType.DMA,
        ],
    )


_s1_scatter = _make_row_scatter(2)
_s2_scatter = _make_row_scatter(4)


def _sc_pool(h3_hbm, cntp_hbm, psum_hbm, pmax_hbm,
             cbuf, cbuf2, rbuf, osum, omax):
    c = lax.axis_index("c")
    s = lax.axis_index("s")
    wid = c * 16 + s
    pltpu.sync_copy(cntp_hbm.at[0], cbuf.at[pl.ds(0, GPAD)])
    pltpu.sync_copy(cntp_hbm.at[1], cbuf2)

    def addc(i, _):
        sl = pl.ds(i * 16, 16)
        cbuf[sl] = cbuf[sl] + cbuf2[sl]
        return 0

    lax.fori_loop(0, GPAD // 16, addc, 0)

    def pre(i, acc16):
        return acc16 + cbuf[pl.ds(i * 16, 16)]

    acc16 = lax.fori_loop(0, wid * 4, pre, _Z16())
    start0 = jnp.sum(acc16).astype(jnp.int32)

    def graph_step(g_local, start):
        g = wid * 64 + g_local
        cnt = cbuf[pl.ds(g, 16)][0].astype(jnp.int32)
        nch = (cnt + 31) // 32

        def chunk_step(k, carry):
            rowstart = jnp.minimum(start + k * 32, NROWS - 32)
            pltpu.sync_copy(h3_hbm.at[pl.ds(rowstart * 256, 32 * 256)], rbuf)
            m = jnp.minimum(32, cnt - k * 32)

            def row_step(r, carry2):
                sums, maxs = carry2
                new_s = []
                new_m = []
                for jj in range(16):
                    v = rbuf[pl.ds(r * 256 + jj * 16, 16)]
                    new_s.append(sums[jj] + v)
                    new_m.append(jnp.maximum(maxs[jj], v))
                return tuple(new_s), tuple(new_m)

            return lax.fori_loop(0, m, row_step, carry)

        init = (tuple(_Z16() for _ in range(16)),
                tuple(jnp.full((16,), NEG_INF, jnp.float32) for _ in range(16)))
        sums, maxs = lax.fori_loop(0, nch, chunk_step, init)
        for jj in range(16):
            osum[g_local, pl.ds(jj * 16, 16)] = sums[jj]
            omax[g_local, pl.ds(jj * 16, 16)] = maxs[jj]
        return start + cnt

    lax.fori_loop(0, 64, graph_step, start0)
    pltpu.sync_copy(osum, psum_hbm.at[pl.ds(wid * 64, 64), :])
    pltpu.sync_copy(omax, pmax_hbm.at[pl.ds(wid * 64, 64), :])


_pool = pl.kernel(
    _sc_pool,
    out_type=[jax.ShapeDtypeStruct((GPAD, 256), jnp.float32),
              jax.ShapeDtypeStruct((GPAD, 256), jnp.float32)],
    mesh=_MESH,
    compiler_params=pltpu.CompilerParams(needs_layout_passes=False),
    scratch_types=[
        pltpu.VMEM((GPAD + 16,), jnp.float32),
        pltpu.VMEM((GPAD,), jnp.float32),
        pltpu.VMEM((32 * 256,), jnp.float32),
        pltpu.VMEM((64, 256), jnp.float32),
        pltpu.VMEM((64, 256), jnp.float32),
    ],
)


_BLK = 1024
_GRID = NROWS // _BLK


def _tc_c0(x_ref, degp_ref, o_dis, o_xt):
    deg = degp_ref[0] + degp_ref[1] + 1.0
    dis = lax.rsqrt(deg)
    o_dis[0, 0] = dis
    o_xt[0, 0] = dis * x_ref[0, 0]


def _c0_call(x2, degp):
    return pl.pallas_call(
        _tc_c0,
        grid=(_GRID,),
        in_specs=[
            pl.BlockSpec((1, 1, _BLK), lambda i: (i, 0, 0)),
            pl.BlockSpec((2, _BLK), lambda i: (0, i)),
        ],
        out_specs=[pl.BlockSpec((1, 1, _BLK), lambda i: (i, 0, 0)),
                   pl.BlockSpec((1, 1, _BLK), lambda i: (i, 0, 0))],
        out_shape=[jax.ShapeDtypeStruct((_GRID, 1, _BLK), jnp.float32),
                   jax.ShapeDtypeStruct((_GRID, 1, _BLK), jnp.float32)],
    )(x2, degp)


def _tc_c1(p0p_ref, xt_ref, dis_ref, w1_ref, b1_ref, o_ht1):
    q = (p0p_ref[0] + p0p_ref[1] + xt_ref[0, 0]) * dis_ref[0, 0]
    h1 = jnp.maximum(q[:, None] * w1_ref[0][None, :] + b1_ref[0][None, :], 0.0)
    ht1 = dis_ref[0, 0][:, None] * h1
    o_ht1[0] = ht1[:, :32]
    o_ht1[1] = ht1[:, 32:]


def _c1_call(p0p, xt2, dis2, W1, b1):
    return pl.pallas_call(
        _tc_c1,
        grid=(_GRID,),
        in_specs=[
            pl.BlockSpec((2, _BLK), lambda i: (0, i)),
            pl.BlockSpec((1, 1, _BLK), lambda i: (i, 0, 0)),
            pl.BlockSpec((1, 1, _BLK), lambda i: (i, 0, 0)),
            pl.BlockSpec((1, 64), lambda i: (0, 0)),
            pl.BlockSpec((1, 64), lambda i: (0, 0)),
        ],
        out_specs=pl.BlockSpec((2, _BLK, 32), lambda i: (0, i, 0)),
        out_shape=jax.ShapeDtypeStruct((2, NROWS, 32), jnp.float32),
    )(p0p, xt2, dis2, W1, b1.reshape(1, 64))


def _tc_e(s1_ref, ht1_ref, dis_ref, w2_ref, b2_ref, o_ht2):
    dis = dis_ref[0, 0]
    q = jnp.concatenate([s1_ref[0] + ht1_ref[0], s1_ref[1] + ht1_ref[1]],
                        axis=1) * dis[:, None]
    h2 = jnp.dot(q, w2_ref[...], preferred_element_type=jnp.float32)
    h2 = jnp.maximum(h2 + b2_ref[0][None, :], 0.0)
    ht2 = dis[:, None] * h2
    for j in range(4):
        o_ht2[j] = ht2[:, j * 32:(j + 1) * 32]


def _e_call(s1, ht1, dis2, W2, b2):
    return pl.pallas_call(
        _tc_e,
        grid=(_GRID,),
        in_specs=[
            pl.BlockSpec((2, _BLK, 32), lambda i: (0, i, 0)),
            pl.BlockSpec((2, _BLK, 32), lambda i: (0, i, 0)),
            pl.BlockSpec((1, 1, _BLK), lambda i: (i, 0, 0)),
            pl.BlockSpec((64, 128), lambda i: (0, 0)),
            pl.BlockSpec((1, 128), lambda i: (0, 0)),
        ],
        out_specs=pl.BlockSpec((4, _BLK, 32), lambda i: (0, i, 0)),
        out_shape=jax.ShapeDtypeStruct((4, NROWS, 32), jnp.float32),
    )(s1, ht1, dis2, W2, b2.reshape(1, 128))


def _tc_g(s2_ref, ht2_ref, dis_ref, w3_ref, b3_ref, o_h3):
    dis = dis_ref[0, 0]
    q = jnp.concatenate([s2_ref[j] + ht2_ref[j] for j in range(4)],
                        axis=1) * dis[:, None]
    h3 = jnp.dot(q, w3_ref[...], preferred_element_type=jnp.float32)
    o_h3[...] = h3 + b3_ref[0][None, :]


def _g_call(s2, ht2, dis2, W3, b3):
    return pl.pallas_call(
        _tc_g,
        grid=(_GRID,),
        in_specs=[
            pl.BlockSpec((4, _BLK, 32), lambda i: (0, i, 0)),
            pl.BlockSpec((4, _BLK, 32), lambda i: (0, i, 0)),
            pl.BlockSpec((1, 1, _BLK), lambda i: (i, 0, 0)),
            pl.BlockSpec((128, 256), lambda i: (0, 0)),
            pl.BlockSpec((1, 256), lambda i: (0, 0)),
        ],
        out_specs=pl.BlockSpec((_BLK, 256), lambda i: (i, 0)),
        out_shape=jax.ShapeDtypeStruct((NROWS, 256), jnp.float32),
    )(s2, ht2, dis2, W3, b3.reshape(1, 256))


def _tc_head(psum_ref, pmax_ref, cntp_ref, fw1_ref, fb1_ref, fw2_ref, fb2_ref,
             o_ref):
    counts = cntp_ref[0, :G] + cntp_ref[1, :G]
    mean = psum_ref[:G] / jnp.maximum(counts, 1.0)[:, None]
    mx = jnp.where(counts[:, None] > 0, pmax_ref[:G], 0.0)
    z = jnp.concatenate([mean, mx], axis=1)
    z = jnp.dot(z, fw1_ref[...], preferred_element_type=jnp.float32)
    z = jnp.maximum(z + fb1_ref[0][None, :], 0.0)
    out = jnp.dot(z, fw2_ref[...], preferred_element_type=jnp.float32)
    o_ref[...] = out + fb2_ref[0][None, :]


def _head_call(psum, pmax, cntp, fW1, fb1, fW2, fb2):
    return pl.pallas_call(
        _tc_head,
        grid=(1,),
        in_specs=[
            pl.BlockSpec((GPAD, 256), lambda i: (0, 0)),
            pl.BlockSpec((GPAD, 256), lambda i: (0, 0)),
            pl.BlockSpec((2, GPAD), lambda i: (0, 0)),
            pl.BlockSpec((512, 128), lambda i: (0, 0)),
            pl.BlockSpec((1, 128), lambda i: (0, 0)),
            pl.BlockSpec((128, 12), lambda i: (0, 0)),
            pl.BlockSpec((1, 12), lambda i: (0, 0)),
        ],
        out_specs=pl.BlockSpec((G, 12), lambda i: (0, 0)),
        out_shape=jax.ShapeDtypeStruct((G, 12), jnp.float32),
    )(psum, pmax, cntp, fW1, fb1.reshape(1, 128), fW2, fb2.reshape(1, 12))


def kernel(x, edge_index, batch, W1, b1, W2, b2, W3, b3, fW1, fb1, fW2, fb2):
    src = edge_index[0]
    dst = edge_index[1]
    bat_pad = jnp.concatenate(
        [batch, jnp.full((NROWS - N,), G, jnp.int32)])
    x2 = jnp.pad(x[:, 0], (0, NROWS - N)).reshape(_GRID, 1, _BLK)

    degp, cntp = _deg_counts(dst, bat_pad)
    dis2, xt2 = _c0_call(x2, degp)
    (p0p,) = _p0_scatter(xt2.reshape(NROWS), src, dst)
    ht1 = _c1_call(p0p, xt2, dis2, W1, b1)
    (s1,) = _s1_scatter(ht1, src, dst)
    ht2 = _e_call(s1, ht1, dis2, W2, b2)
    (s2,) = _s2_scatter(ht2, src, dst)
    h3 = _g_call(s2, ht2, dis2, W3, b3)
    psum, pmax = _pool(h3.reshape(NROWS * 256), cntp)
    return _head_call(psum, pmax, cntp, fW1, fb1, fW2, fb2)


# speedup vs baseline: 15.2641x; 1.4608x over previous
import functools
import jax
import jax.numpy as jnp
from jax import lax
from jax.experimental import pallas as pl
from jax.experimental.pallas import tpu as pltpu
from jax.experimental.pallas import tpu_sc as plsc

N = 50000
E = 800000
G = 2000
NPAD = 51200
NROWS = 50176
GPAD = 2048
ECH = E // 128
BCH = NROWS // 128
NEG_INF = float("-inf")

_MESH = plsc.VectorSubcoreMesh(core_axis_name="c", subcore_axis_name="s")
_Z16 = functools.partial(jnp.zeros, (16,), jnp.float32)


def _fill(ref, n, value):
    v = jnp.full((16,), value, jnp.float32)

    def body(i, _):
        ref[pl.ds(i * 16, 16)] = v
        return 0

    lax.fori_loop(0, n // 16, body, 0)


def _fill2d(ref, rows, value):
    v = jnp.full((16,), value, jnp.float32)

    def body(i, _):
        r = i // 2
        col = (i % 2) * 16
        ref[r, pl.ds(col, 16)] = v
        return 0

    lax.fori_loop(0, rows * 2, body, 0)


def _sc_deg_counts(dst_hbm, bat_hbm, degp_hbm, cntp_hbm,
                   accd, accc, zbuf, obuf, idx):
    c = lax.axis_index("c")
    s = lax.axis_index("s")
    w = c * 16 + s
    _fill(zbuf, 3200, 0.0)
    _fill(obuf, 128, 1.0)
    pltpu.sync_copy(zbuf, accd.at[pl.ds(s * 3200, 3200)])
    pltpu.sync_copy(zbuf.at[pl.ds(0, 128)], accc.at[pl.ds(s * 128, 128)])
    plsc.subcore_barrier()

    def estep(k, _):
        pltpu.sync_copy(dst_hbm.at[pl.ds(k * 128, 128)], idx.at[0])
        pltpu.sync_copy(obuf, accd.at[idx.at[0]], add=True)
        return 0

    lax.fori_loop(w * ECH // 32, (w + 1) * ECH // 32, estep, 0)

    def bstep(k, _):
        pltpu.sync_copy(bat_hbm.at[pl.ds(k * 128, 128)], idx.at[0])
        pltpu.sync_copy(obuf, accc.at[idx.at[0]], add=True)
        return 0

    lax.fori_loop(w * BCH // 32, (w + 1) * BCH // 32, bstep, 0)

    plsc.subcore_barrier()
    pltpu.sync_copy(accd.at[pl.ds(s * 3200, 3200)],
                    degp_hbm.at[c].at[pl.ds(s * 3200, 3200)])
    pltpu.sync_copy(accc.at[pl.ds(s * 128, 128)],
                    cntp_hbm.at[c].at[pl.ds(s * 128, 128)])


_deg_counts = pl.kernel(
    _sc_deg_counts,
    out_type=[jax.ShapeDtypeStruct((2, NPAD), jnp.float32),
              jax.ShapeDtypeStruct((2, GPAD), jnp.float32)],
    mesh=_MESH,
    scratch_types=[
        pltpu.VMEM_SHARED((NPAD,), jnp.float32),
        pltpu.VMEM_SHARED((GPAD,), jnp.float32),
        pltpu.VMEM((3200,), jnp.float32),
        pltpu.VMEM((128,), jnp.float32),
        pltpu.VMEM((1, 128), jnp.int32),
    ],
)


def _sc_p0(xt_hbm, src_hbm, dst_hbm, p0p_hbm,
           acc, zbuf, vals, idxs, idxd, sem):
    c = lax.axis_index("c")
    s = lax.axis_index("s")
    w = c * 16 + s
    _fill(zbuf, 3200, 0.0)
    pltpu.sync_copy(zbuf, acc.at[pl.ds(s * 3200, 3200)])
    plsc.subcore_barrier()

    def estep(k, _):
        b = k * 128
        pltpu.sync_copy(src_hbm.at[pl.ds(b, 128)], idxs.at[0])
        pltpu.async_copy(xt_hbm.at[idxs.at[0]], vals, sem).wait()
        pltpu.sync_copy(dst_hbm.at[pl.ds(b, 128)], idxd.at[0])
        pltpu.sync_copy(vals, acc.at[idxd.at[0]], add=True)
        return 0

    lax.fori_loop(w * ECH // 32, (w + 1) * ECH // 32, estep, 0)

    plsc.subcore_barrier()
    pltpu.sync_copy(acc.at[pl.ds(s * 3200, 3200)],
                    p0p_hbm.at[c].at[pl.ds(s * 3200, 3200)])


_p0_scatter = pl.kernel(
    _sc_p0,
    out_type=[jax.ShapeDtypeStruct((2, NPAD), jnp.float32)],
    mesh=_MESH,
    scratch_types=[
        pltpu.VMEM_SHARED((NPAD,), jnp.float32),
        pltpu.VMEM((3200,), jnp.float32),
        pltpu.VMEM((128,), jnp.float32),
        pltpu.VMEM((1, 128), jnp.int32),
        pltpu.VMEM((1, 128), jnp.int32),
        pltpu.SemaphoreType.DMA,
    ],
)


def _make_row_scatter(num_chunks):
    per_core = num_chunks // 2
    KB = 2
    NSB = 96
    MAIN = 16 * 384
    TAIL = ECH - MAIN

    def body(ht_hbm, src2_hbm, dst2_hbm, out_hbm,
             acc, zrows, rows0, rows1, idxs0, idxs1, idxd0, idxd1,
             sg0, sg1, ss0, ss1, st):
        c = lax.axis_index("c")
        s = lax.axis_index("s")
        _fill2d(zrows, 80, 0.0)
        rows = (rows0, rows1)
        idxs = (idxs0, idxs1)
        idxd = (idxd0, idxd1)
        sg = (sg0, sg1)
        ss = (ss0, ss1)

        for j in range(per_core):
            ch = c * per_core + j

            def zstep(i, _):
                pltpu.sync_copy(zrows, acc.at[pl.ds(s * 3200 + i * 80, 80), :])
                return 0

            lax.fori_loop(0, 40, zstep, 0)
            plsc.subcore_barrier()

            def drain_scatters(q):
                for b in range(KB):
                    pltpu.make_async_copy(
                        rows[q].at[pl.ds(b * 128, 128), :],
                        acc.at[idxd[q].at[b]], ss[q]).wait()

            def do_block(sb, q):
                row0 = s * 384 + (2 * sb + q) * KB

                @pl.when(sb > 0)
                def _():
                    drain_scatters(q)

                pltpu.sync_copy(src2_hbm.at[pl.ds(row0, KB), :], idxs[q])
                pltpu.sync_copy(dst2_hbm.at[pl.ds(row0, KB), :], idxd[q])
                for b in range(KB):
                    pltpu.async_copy(ht_hbm.at[ch].at[idxs[q].at[b]],
                                     rows[q].at[pl.ds(b * 128, 128), :], sg[q])
                for b in range(KB):
                    pltpu.make_async_copy(ht_hbm.at[ch].at[idxs[q].at[b]],
                                          rows[q].at[pl.ds(b * 128, 128), :],
                                          sg[q]).wait()
                for b in range(KB):
                    pltpu.async_copy(rows[q].at[pl.ds(b * 128, 128), :],
                                     acc.at[idxd[q].at[b]], ss[q], add=True)

            def sbstep(sb, _):
                do_block(sb, 0)
                do_block(sb, 1)
                return 0

            lax.fori_loop(0, NSB, sbstep, 0)
            drain_scatters(0)
            drain_scatters(1)

            def tstep(k, _):
                pltpu.sync_copy(src2_hbm.at[pl.ds(k, 1), :],
                                idxs0.at[pl.ds(0, 1), :])
                pltpu.async_copy(ht_hbm.at[ch].at[idxs0.at[0]],
                                 rows0.at[pl.ds(0, 128), :], st).wait()
                pltpu.sync_copy(dst2_hbm.at[pl.ds(k, 1), :],
                                idxd0.at[pl.ds(0, 1), :])
                pltpu.sync_copy(rows0.at[pl.ds(0, 128), :],
                                acc.at[idxd0.at[0]], add=True)
                return 0

            lax.fori_loop(MAIN + s * TAIL // 16, MAIN + (s + 1) * TAIL // 16,
                          tstep, 0)

            plsc.subcore_barrier()
            pltpu.sync_copy(acc.at[pl.ds(s * 3200, 3200), :],
                            out_hbm.at[ch].at[pl.ds(s * 3200, 3200), :])
            plsc.subcore_barrier()

    return pl.kernel(
        body,
        out_type=[jax.ShapeDtypeStruct((num_chunks, NPAD, 32), jnp.float32)],
        mesh=_MESH,
        compiler_params=pltpu.CompilerParams(use_tc_tiling_on_sc=False),
        scratch_types=[
            pltpu.VMEM_SHARED((NPAD, 32), jnp.float32),
            pltpu.VMEM((80, 32), jnp.float32),
            pltpu.VMEM((256, 32), jnp.float32),
            pltpu.VMEM((256, 32), jnp.float32),
            pltpu.VMEM((2, 128), jnp.int32),
            pltpu.VMEM((2, 128), jnp.int32),
            pltpu.VMEM((2, 128), jnp.int32),
            pltpu.VMEM((2, 128), jnp.int32),
            pltpu.SemaphoreType.DMA,
            pltpu.SemaphoreType.DMA,
            pltpu.SemaphoreType.DMA,
            pltpu.SemaphoreType.DMA,
            pltpu.SemaphoreType.DMA,
        ],
    )


_s1_scatter = _make_row_scatter(2)
_s2_scatter = _make_row_scatter(4)


def _sc_pool(h3_hbm, cntp_hbm, psum_hbm, pmax_hbm,
             cbuf, cbuf2, rbuf, osum, omax):
    c = lax.axis_index("c")
    s = lax.axis_index("s")
    wid = c * 16 + s
    pltpu.sync_copy(cntp_hbm.at[0], cbuf.at[pl.ds(0, GPAD)])
    pltpu.sync_copy(cntp_hbm.at[1], cbuf2)

    def addc(i, _):
        sl = pl.ds(i * 16, 16)
        cbuf[sl] = cbuf[sl] + cbuf2[sl]
        return 0

    lax.fori_loop(0, GPAD // 16, addc, 0)

    def pre(i, acc16):
        return acc16 + cbuf[pl.ds(i * 16, 16)]

    acc16 = lax.fori_loop(0, wid * 4, pre, _Z16())
    start0 = jnp.sum(acc16).astype(jnp.int32)

    def graph_step(g_local, start):
        g = wid * 64 + g_local
        cnt = cbuf[pl.ds(g, 16)][0].astype(jnp.int32)
        nch = (cnt + 31) // 32

        def chunk_step(k, carry):
            rowstart = jnp.minimum(start + k * 32, NROWS - 32)
            pltpu.sync_copy(h3_hbm.at[pl.ds(rowstart * 256, 32 * 256)], rbuf)
            m = jnp.minimum(32, cnt - k * 32)

            def row_step(r, carry2):
                sums, maxs = carry2
                new_s = []
                new_m = []
                for jj in range(16):
                    v = rbuf[pl.ds(r * 256 + jj * 16, 16)]
                    new_s.append(sums[jj] + v)
                    new_m.append(jnp.maximum(maxs[jj], v))
                return tuple(new_s), tuple(new_m)

            return lax.fori_loop(0, m, row_step, carry)

        init = (tuple(_Z16() for _ in range(16)),
                tuple(jnp.full((16,), NEG_INF, jnp.float32) for _ in range(16)))
        sums, maxs = lax.fori_loop(0, nch, chunk_step, init)
        for jj in range(16):
            osum[g_local, pl.ds(jj * 16, 16)] = sums[jj]
            omax[g_local, pl.ds(jj * 16, 16)] = maxs[jj]
        return start + cnt

    lax.fori_loop(0, 64, graph_step, start0)
    pltpu.sync_copy(osum, psum_hbm.at[pl.ds(wid * 64, 64), :])
    pltpu.sync_copy(omax, pmax_hbm.at[pl.ds(wid * 64, 64), :])


_pool = pl.kernel(
    _sc_pool,
    out_type=[jax.ShapeDtypeStruct((GPAD, 256), jnp.float32),
              jax.ShapeDtypeStruct((GPAD, 256), jnp.float32)],
    mesh=_MESH,
    compiler_params=pltpu.CompilerParams(needs_layout_passes=False),
    scratch_types=[
        pltpu.VMEM((GPAD + 16,), jnp.float32),
        pltpu.VMEM((GPAD,), jnp.float32),
        pltpu.VMEM((32 * 256,), jnp.float32),
        pltpu.VMEM((64, 256), jnp.float32),
        pltpu.VMEM((64, 256), jnp.float32),
    ],
)


_BLK = 1024
_GRID = NROWS // _BLK


def _tc_c0(x_ref, degp_ref, o_dis, o_xt):
    deg = degp_ref[0] + degp_ref[1] + 1.0
    dis = lax.rsqrt(deg)
    o_dis[0, 0] = dis
    o_xt[0, 0] = dis * x_ref[0, 0]


def _c0_call(x2, degp):
    return pl.pallas_call(
        _tc_c0,
        grid=(_GRID,),
        in_specs=[
            pl.BlockSpec((1, 1, _BLK), lambda i: (i, 0, 0)),
            pl.BlockSpec((2, _BLK), lambda i: (0, i)),
        ],
        out_specs=[pl.BlockSpec((1, 1, _BLK), lambda i: (i, 0, 0)),
                   pl.BlockSpec((1, 1, _BLK), lambda i: (i, 0, 0))],
        out_shape=[jax.ShapeDtypeStruct((_GRID, 1, _BLK), jnp.float32),
                   jax.ShapeDtypeStruct((_GRID, 1, _BLK), jnp.float32)],
    )(x2, degp)


def _tc_c1(p0p_ref, xt_ref, dis_ref, w1_ref, b1_ref, o_ht1):
    q = (p0p_ref[0] + p0p_ref[1] + xt_ref[0, 0]) * dis_ref[0, 0]
    h1 = jnp.maximum(q[:, None] * w1_ref[0][None, :] + b1_ref[0][None, :], 0.0)
    ht1 = dis_ref[0, 0][:, None] * h1
    o_ht1[0] = ht1[:, :32]
    o_ht1[1] = ht1[:, 32:]


def _c1_call(p0p, xt2, dis2, W1, b1):
    return pl.pallas_call(
        _tc_c1,
        grid=(_GRID,),
        in_specs=[
            pl.BlockSpec((2, _BLK), lambda i: (0, i)),
            pl.BlockSpec((1, 1, _BLK), lambda i: (i, 0, 0)),
            pl.BlockSpec((1, 1, _BLK), lambda i: (i, 0, 0)),
            pl.BlockSpec((1, 64), lambda i: (0, 0)),
            pl.BlockSpec((1, 64), lambda i: (0, 0)),
        ],
        out_specs=pl.BlockSpec((2, _BLK, 32), lambda i: (0, i, 0)),
        out_shape=jax.ShapeDtypeStruct((2, NROWS, 32), jnp.float32),
    )(p0p, xt2, dis2, W1, b1.reshape(1, 64))


def _tc_e(s1_ref, ht1_ref, dis_ref, w2_ref, b2_ref, o_ht2):
    dis = dis_ref[0, 0]
    q = jnp.concatenate([s1_ref[0] + ht1_ref[0], s1_ref[1] + ht1_ref[1]],
                        axis=1) * dis[:, None]
    h2 = jnp.dot(q, w2_ref[...], preferred_element_type=jnp.float32)
    h2 = jnp.maximum(h2 + b2_ref[0][None, :], 0.0)
    ht2 = dis[:, None] * h2
    for j in range(4):
        o_ht2[j] = ht2[:, j * 32:(j + 1) * 32]


def _e_call(s1, ht1, dis2, W2, b2):
    return pl.pallas_call(
        _tc_e,
        grid=(_GRID,),
        in_specs=[
            pl.BlockSpec((2, _BLK, 32), lambda i: (0, i, 0)),
            pl.BlockSpec((2, _BLK, 32), lambda i: (0, i, 0)),
            pl.BlockSpec((1, 1, _BLK), lambda i: (i, 0, 0)),
            pl.BlockSpec((64, 128), lambda i: (0, 0)),
            pl.BlockSpec((1, 128), lambda i: (0, 0)),
        ],
        out_specs=pl.BlockSpec((4, _BLK, 32), lambda i: (0, i, 0)),
        out_shape=jax.ShapeDtypeStruct((4, NROWS, 32), jnp.float32),
    )(s1, ht1, dis2, W2, b2.reshape(1, 128))


def _tc_g(s2_ref, ht2_ref, dis_ref, w3_ref, b3_ref, o_h3):
    dis = dis_ref[0, 0]
    q = jnp.concatenate([s2_ref[j] + ht2_ref[j] for j in range(4)],
                        axis=1) * dis[:, None]
    h3 = jnp.dot(q, w3_ref[...], preferred_element_type=jnp.float32)
    o_h3[...] = h3 + b3_ref[0][None, :]


def _g_call(s2, ht2, dis2, W3, b3):
    return pl.pallas_call(
        _tc_g,
        grid=(_GRID,),
        in_specs=[
            pl.BlockSpec((4, _BLK, 32), lambda i: (0, i, 0)),
            pl.BlockSpec((4, _BLK, 32), lambda i: (0, i, 0)),
            pl.BlockSpec((1, 1, _BLK), lambda i: (i, 0, 0)),
            pl.BlockSpec((128, 256), lambda i: (0, 0)),
            pl.BlockSpec((1, 256), lambda i: (0, 0)),
        ],
        out_specs=pl.BlockSpec((_BLK, 256), lambda i: (i, 0)),
        out_shape=jax.ShapeDtypeStruct((NROWS, 256), jnp.float32),
    )(s2, ht2, dis2, W3, b3.reshape(1, 256))


def _tc_head(psum_ref, pmax_ref, cntp_ref, fw1_ref, fb1_ref, fw2_ref, fb2_ref,
             o_ref):
    counts = cntp_ref[0, :G] + cntp_ref[1, :G]
    mean = psum_ref[:G] / jnp.maximum(counts, 1.0)[:, None]
    mx = jnp.where(counts[:, None] > 0, pmax_ref[:G], 0.0)
    z = jnp.concatenate([mean, mx], axis=1)
    z = jnp.dot(z, fw1_ref[...], preferred_element_type=jnp.float32)
    z = jnp.maximum(z + fb1_ref[0][None, :], 0.0)
    out = jnp.dot(z, fw2_ref[...], preferred_element_type=jnp.float32)
    o_ref[...] = out + fb2_ref[0][None, :]


def _head_call(psum, pmax, cntp, fW1, fb1, fW2, fb2):
    return pl.pallas_call(
        _tc_head,
        grid=(1,),
        in_specs=[
            pl.BlockSpec((GPAD, 256), lambda i: (0, 0)),
            pl.BlockSpec((GPAD, 256), lambda i: (0, 0)),
            pl.BlockSpec((2, GPAD), lambda i: (0, 0)),
            pl.BlockSpec((512, 128), lambda i: (0, 0)),
            pl.BlockSpec((1, 128), lambda i: (0, 0)),
            pl.BlockSpec((128, 12), lambda i: (0, 0)),
            pl.BlockSpec((1, 12), lambda i: (0, 0)),
        ],
        out_specs=pl.BlockSpec((G, 12), lambda i: (0, 0)),
        out_shape=jax.ShapeDtypeStruct((G, 12), jnp.float32),
    )(psum, pmax, cntp, fW1, fb1.reshape(1, 128), fW2, fb2.reshape(1, 12))


def kernel(x, edge_index, batch, W1, b1, W2, b2, W3, b3, fW1, fb1, fW2, fb2):
    src = edge_index[0]
    dst = edge_index[1]
    bat_pad = jnp.concatenate(
        [batch, jnp.full((NROWS - N,), G, jnp.int32)])
    x2 = jnp.pad(x[:, 0], (0, NROWS - N)).reshape(_GRID, 1, _BLK)

    degp, cntp = _deg_counts(dst, bat_pad)
    dis2, xt2 = _c0_call(x2, degp)
    (p0p,) = _p0_scatter(xt2.reshape(NROWS), src, dst)
    ht1 = _c1_call(p0p, xt2, dis2, W1, b1)
    src2 = src.reshape(ECH, 128)
    dst2 = dst.reshape(ECH, 128)
    (s1,) = _s1_scatter(ht1, src2, dst2)
    ht2 = _e_call(s1, ht1, dis2, W2, b2)
    (s2,) = _s2_scatter(ht2, src2, dst2)
    h3 = _g_call(s2, ht2, dis2, W3, b3)
    psum, pmax = _pool(h3.reshape(NROWS * 256), cntp)
    return _head_call(psum, pmax, cntp, fW1, fb1, fW2, fb2)


# speedup vs baseline: 23.1516x; 1.5167x over previous
import functools
import jax
import jax.numpy as jnp
from jax import lax
from jax.experimental import pallas as pl
from jax.experimental.pallas import tpu as pltpu
from jax.experimental.pallas import tpu_sc as plsc

N = 50000
E = 800000
G = 2000
NPAD = 51200
NROWS = 50176
GPAD = 2048
ECH = E // 128
BCH = NROWS // 128
NEG_INF = float("-inf")

_MESH = plsc.VectorSubcoreMesh(core_axis_name="c", subcore_axis_name="s")
_Z16 = functools.partial(jnp.zeros, (16,), jnp.float32)


def _fill(ref, n, value):
    v = jnp.full((16,), value, jnp.float32)

    def body(i, _):
        ref[pl.ds(i * 16, 16)] = v
        return 0

    lax.fori_loop(0, n // 16, body, 0)


def _fill2d(ref, rows, value):
    v = jnp.full((16,), value, jnp.float32)

    def body(i, _):
        r = i // 2
        col = (i % 2) * 16
        ref[r, pl.ds(col, 16)] = v
        return 0

    lax.fori_loop(0, rows * 2, body, 0)


def _sc_deg_counts(dst_hbm, bat_hbm, degp_hbm, cntp_hbm,
                   accd, accc, zbuf, obuf, idx):
    c = lax.axis_index("c")
    s = lax.axis_index("s")
    w = c * 16 + s
    _fill(zbuf, 3200, 0.0)
    _fill(obuf, 128, 1.0)
    pltpu.sync_copy(zbuf, accd.at[pl.ds(s * 3200, 3200)])
    pltpu.sync_copy(zbuf.at[pl.ds(0, 128)], accc.at[pl.ds(s * 128, 128)])
    plsc.subcore_barrier()

    def estep(k, _):
        pltpu.sync_copy(dst_hbm.at[pl.ds(k * 128, 128)], idx.at[0])
        pltpu.sync_copy(obuf, accd.at[idx.at[0]], add=True)
        return 0

    lax.fori_loop(w * ECH // 32, (w + 1) * ECH // 32, estep, 0)

    def bstep(k, _):
        pltpu.sync_copy(bat_hbm.at[pl.ds(k * 128, 128)], idx.at[0])
        pltpu.sync_copy(obuf, accc.at[idx.at[0]], add=True)
        return 0

    lax.fori_loop(w * BCH // 32, (w + 1) * BCH // 32, bstep, 0)

    plsc.subcore_barrier()
    pltpu.sync_copy(accd.at[pl.ds(s * 3200, 3200)],
                    degp_hbm.at[c].at[pl.ds(s * 3200, 3200)])
    pltpu.sync_copy(accc.at[pl.ds(s * 128, 128)],
                    cntp_hbm.at[c].at[pl.ds(s * 128, 128)])


_deg_counts = pl.kernel(
    _sc_deg_counts,
    out_type=[jax.ShapeDtypeStruct((2, NPAD), jnp.float32),
              jax.ShapeDtypeStruct((2, GPAD), jnp.float32)],
    mesh=_MESH,
    scratch_types=[
        pltpu.VMEM_SHARED((NPAD,), jnp.float32),
        pltpu.VMEM_SHARED((GPAD,), jnp.float32),
        pltpu.VMEM((3200,), jnp.float32),
        pltpu.VMEM((128,), jnp.float32),
        pltpu.VMEM((1, 128), jnp.int32),
    ],
)


def _sc_p0(xt_hbm, src2_hbm, dst2_hbm, p0p_hbm,
           acc, zbuf, vals0, vals1, vals2,
           idxs0, idxs1, idxs2, idxd0, idxd1, idxd2,
           sg0, sg1, sg2, ss0, ss1, ss2, st):
    c = lax.axis_index("c")
    s = lax.axis_index("s")
    w = c * 16 + s
    KB = 8
    NSB = 8
    MAIN = 32 * 192
    TAIL = ECH - MAIN
    _fill(zbuf, 3200, 0.0)
    pltpu.sync_copy(zbuf, acc.at[pl.ds(s * 3200, 3200)])
    plsc.subcore_barrier()
    vals = (vals0, vals1, vals2)
    idxs = (idxs0, idxs1, idxs2)
    idxd = (idxd0, idxd1, idxd2)
    sg = (sg0, sg1, sg2)
    ss = (ss0, ss1, ss2)

    def drain_scatters(q):
        for b in range(KB):
            pltpu.make_async_copy(vals[q].at[pl.ds(b * 128, 128)],
                                  acc.at[idxd[q].at[b]], ss[q]).wait()

    def drain_gathers(q):
        for b in range(KB):
            pltpu.make_async_copy(xt_hbm.at[idxs[q].at[b]],
                                  vals[q].at[pl.ds(b * 128, 128)], sg[q]).wait()

    def fire_scatters(q):
        for b in range(KB):
            pltpu.async_copy(vals[q].at[pl.ds(b * 128, 128)],
                             acc.at[idxd[q].at[b]], ss[q], add=True)

    def do_block(sb, q):
        qp = (q - 1) % 3
        row0 = w * 192 + (3 * sb + q) * KB
        @pl.when(sb > 0)
        def _():
            drain_scatters(q)
        pltpu.sync_copy(src2_hbm.at[pl.ds(row0, KB), :], idxs[q])
        pltpu.sync_copy(dst2_hbm.at[pl.ds(row0, KB), :], idxd[q])
        for b in range(KB):
            pltpu.async_copy(xt_hbm.at[idxs[q].at[b]],
                             vals[q].at[pl.ds(b * 128, 128)], sg[q])
        if q == 0:
            @pl.when(sb > 0)
            def _():
                drain_gathers(qp)
                fire_scatters(qp)
        else:
            drain_gathers(qp)
            fire_scatters(qp)

    def sbstep(sb, _):
        do_block(sb, 0)
        do_block(sb, 1)
        do_block(sb, 2)
        return 0

    lax.fori_loop(0, NSB, sbstep, 0)
    drain_gathers(2)
    fire_scatters(2)
    drain_scatters(0)
    drain_scatters(1)
    drain_scatters(2)

    def tstep(k, _):
        pltpu.sync_copy(src2_hbm.at[pl.ds(k, 1), :], idxs0.at[pl.ds(0, 1), :])
        pltpu.async_copy(xt_hbm.at[idxs0.at[0]],
                         vals0.at[pl.ds(0, 128)], st).wait()
        pltpu.sync_copy(dst2_hbm.at[pl.ds(k, 1), :], idxd0.at[pl.ds(0, 1), :])
        pltpu.sync_copy(vals0.at[pl.ds(0, 128)],
                        acc.at[idxd0.at[0]], add=True)
        return 0

    lax.fori_loop(MAIN + w * TAIL // 32, MAIN + (w + 1) * TAIL // 32, tstep, 0)

    plsc.subcore_barrier()
    pltpu.sync_copy(acc.at[pl.ds(s * 3200, 3200)],
                    p0p_hbm.at[c].at[pl.ds(s * 3200, 3200)])


_p0_scatter = pl.kernel(
    _sc_p0,
    out_type=[jax.ShapeDtypeStruct((2, NPAD), jnp.float32)],
    mesh=_MESH,
    compiler_params=pltpu.CompilerParams(use_tc_tiling_on_sc=False),
    scratch_types=[
        pltpu.VMEM_SHARED((NPAD,), jnp.float32),
        pltpu.VMEM((3200,), jnp.float32),
        pltpu.VMEM((1024,), jnp.float32),
        pltpu.VMEM((1024,), jnp.float32),
        pltpu.VMEM((1024,), jnp.float32),
        pltpu.VMEM((8, 128), jnp.int32),
        pltpu.VMEM((8, 128), jnp.int32),
        pltpu.VMEM((8, 128), jnp.int32),
        pltpu.VMEM((8, 128), jnp.int32),
        pltpu.VMEM((8, 128), jnp.int32),
        pltpu.VMEM((8, 128), jnp.int32),
        pltpu.SemaphoreType.DMA,
        pltpu.SemaphoreType.DMA,
        pltpu.SemaphoreType.DMA,
        pltpu.SemaphoreType.DMA,
        pltpu.SemaphoreType.DMA,
        pltpu.SemaphoreType.DMA,
        pltpu.SemaphoreType.DMA,
    ],
)


def _make_row_scatter(num_chunks):
    per_core = num_chunks // 2
    KB = 2
    NSB = 64
    MAIN = 16 * 384
    TAIL = ECH - MAIN

    def body(ht_hbm, src2_hbm, dst2_hbm, out_hbm,
             acc, zrows, rows0, rows1, rows2,
             idxs0, idxs1, idxs2, idxd0, idxd1, idxd2,
             sg0, sg1, sg2, ss0, ss1, ss2, st):
        c = lax.axis_index("c")
        s = lax.axis_index("s")
        _fill2d(zrows, 72, 0.0)
        rows = (rows0, rows1, rows2)
        idxs = (idxs0, idxs1, idxs2)
        idxd = (idxd0, idxd1, idxd2)
        sg = (sg0, sg1, sg2)
        ss = (ss0, ss1, ss2)

        for j in range(per_core):
            ch = c * per_core + j

            def zstep(i, _):
                pltpu.sync_copy(zrows, acc.at[pl.ds(s * 3200 + i * 72, 72), :])
                return 0

            lax.fori_loop(0, 44, zstep, 0)
            pltpu.sync_copy(zrows.at[pl.ds(0, 32), :],
                            acc.at[pl.ds(s * 3200 + 3168, 32), :])
            plsc.subcore_barrier()

            def drain_scatters(q):
                for b in range(KB):
                    pltpu.make_async_copy(
                        rows[q].at[pl.ds(b * 128, 128), :],
                        acc.at[idxd[q].at[b]], ss[q]).wait()

            def drain_gathers(q):
                for b in range(KB):
                    pltpu.make_async_copy(
                        ht_hbm.at[ch].at[idxs[q].at[b]],
                        rows[q].at[pl.ds(b * 128, 128), :], sg[q]).wait()

            def fire_scatters(q):
                for b in range(KB):
                    pltpu.async_copy(rows[q].at[pl.ds(b * 128, 128), :],
                                     acc.at[idxd[q].at[b]], ss[q], add=True)

            def do_block(sb, q):
                qp = (q - 1) % 3
                row0 = s * 384 + (3 * sb + q) * KB

                @pl.when(sb > 0)
                def _():
                    drain_scatters(q)

                pltpu.sync_copy(src2_hbm.at[pl.ds(row0, KB), :], idxs[q])
                pltpu.sync_copy(dst2_hbm.at[pl.ds(row0, KB), :], idxd[q])
                for b in range(KB):
                    pltpu.async_copy(ht_hbm.at[ch].at[idxs[q].at[b]],
                                     rows[q].at[pl.ds(b * 128, 128), :], sg[q])
                if q == 0:
                    @pl.when(sb > 0)
                    def _():
                        drain_gathers(qp)
                        fire_scatters(qp)
                else:
                    drain_gathers(qp)
                    fire_scatters(qp)

            def sbstep(sb, _):
                do_block(sb, 0)
                do_block(sb, 1)
                do_block(sb, 2)
                return 0

            lax.fori_loop(0, NSB, sbstep, 0)
            drain_gathers(2)
            fire_scatters(2)
            drain_scatters(0)
            drain_scatters(1)
            drain_scatters(2)

            def tstep(k, _):
                pltpu.sync_copy(src2_hbm.at[pl.ds(k, 1), :],
                                idxs0.at[pl.ds(0, 1), :])
                pltpu.async_copy(ht_hbm.at[ch].at[idxs0.at[0]],
                                 rows0.at[pl.ds(0, 128), :], st).wait()
                pltpu.sync_copy(dst2_hbm.at[pl.ds(k, 1), :],
                                idxd0.at[pl.ds(0, 1), :])
                pltpu.sync_copy(rows0.at[pl.ds(0, 128), :],
                                acc.at[idxd0.at[0]], add=True)
                return 0

            lax.fori_loop(MAIN + s * TAIL // 16, MAIN + (s + 1) * TAIL // 16,
                          tstep, 0)

            plsc.subcore_barrier()
            pltpu.sync_copy(acc.at[pl.ds(s * 3200, 3200), :],
                            out_hbm.at[ch].at[pl.ds(s * 3200, 3200), :])
            plsc.subcore_barrier()

    return pl.kernel(
        body,
        out_type=[jax.ShapeDtypeStruct((num_chunks, NPAD, 32), jnp.float32)],
        mesh=_MESH,
        compiler_params=pltpu.CompilerParams(use_tc_tiling_on_sc=False),
        scratch_types=[
            pltpu.VMEM_SHARED((NPAD, 32), jnp.float32),
            pltpu.VMEM((72, 32), jnp.float32),
            pltpu.VMEM((256, 32), jnp.float32),
            pltpu.VMEM((256, 32), jnp.float32),
            pltpu.VMEM((256, 32), jnp.float32),
            pltpu.VMEM((2, 128), jnp.int32),
            pltpu.VMEM((2, 128), jnp.int32),
            pltpu.VMEM((2, 128), jnp.int32),
            pltpu.VMEM((2, 128), jnp.int32),
            pltpu.VMEM((2, 128), jnp.int32),
            pltpu.VMEM((2, 128), jnp.int32),
            pltpu.SemaphoreType.DMA,
            pltpu.SemaphoreType.DMA,
            pltpu.SemaphoreType.DMA,
            pltpu.SemaphoreType.DMA,
            pltpu.SemaphoreType.DMA,
            pltpu.SemaphoreType.DMA,
            pltpu.SemaphoreType.DMA,
        ],
    )


_s1_scatter = _make_row_scatter(2)
_s2_scatter = _make_row_scatter(4)


def _sc_pool(h3_hbm, cntp_hbm, psum_hbm, pmax_hbm,
             cbuf, cbuf2, rbuf, osum, omax):
    c = lax.axis_index("c")
    s = lax.axis_index("s")
    wid = c * 16 + s
    pltpu.sync_copy(cntp_hbm.at[0], cbuf.at[pl.ds(0, GPAD)])
    pltpu.sync_copy(cntp_hbm.at[1], cbuf2)

    def addc(i, _):
        sl = pl.ds(i * 16, 16)
        cbuf[sl] = cbuf[sl] + cbuf2[sl]
        return 0

    lax.fori_loop(0, GPAD // 16, addc, 0)

    def pre(i, acc16):
        return acc16 + cbuf[pl.ds(i * 16, 16)]

    acc16 = lax.fori_loop(0, wid * 4, pre, _Z16())
    start0 = jnp.sum(acc16).astype(jnp.int32)

    def graph_step(g_local, start):
        g = wid * 64 + g_local
        cnt = cbuf[pl.ds(g, 16)][0].astype(jnp.int32)
        nch = (cnt + 31) // 32

        def chunk_step(k, carry):
            rowstart = jnp.minimum(start + k * 32, NROWS - 32)
            pltpu.sync_copy(h3_hbm.at[pl.ds(rowstart * 256, 32 * 256)], rbuf)
            m = jnp.minimum(32, cnt - k * 32)

            def row_step(r, carry2):
                sums, maxs = carry2
                new_s = []
                new_m = []
                for jj in range(16):
                    v = rbuf[pl.ds(r * 256 + jj * 16, 16)]
                    new_s.append(sums[jj] + v)
                    new_m.append(jnp.maximum(maxs[jj], v))
                return tuple(new_s), tuple(new_m)

            return lax.fori_loop(0, m, row_step, carry)

        init = (tuple(_Z16() for _ in range(16)),
                tuple(jnp.full((16,), NEG_INF, jnp.float32) for _ in range(16)))
        sums, maxs = lax.fori_loop(0, nch, chunk_step, init)
        for jj in range(16):
            osum[g_local, pl.ds(jj * 16, 16)] = sums[jj]
            omax[g_local, pl.ds(jj * 16, 16)] = maxs[jj]
        return start + cnt

    lax.fori_loop(0, 64, graph_step, start0)
    pltpu.sync_copy(osum, psum_hbm.at[pl.ds(wid * 64, 64), :])
    pltpu.sync_copy(omax, pmax_hbm.at[pl.ds(wid * 64, 64), :])


_pool = pl.kernel(
    _sc_pool,
    out_type=[jax.ShapeDtypeStruct((GPAD, 256), jnp.float32),
              jax.ShapeDtypeStruct((GPAD, 256), jnp.float32)],
    mesh=_MESH,
    compiler_params=pltpu.CompilerParams(needs_layout_passes=False),
    scratch_types=[
        pltpu.VMEM((GPAD + 16,), jnp.float32),
        pltpu.VMEM((GPAD,), jnp.float32),
        pltpu.VMEM((32 * 256,), jnp.float32),
        pltpu.VMEM((64, 256), jnp.float32),
        pltpu.VMEM((64, 256), jnp.float32),
    ],
)


_BLK = 1024
_GRID = NROWS // _BLK


def _tc_c0(x_ref, degp_ref, o_dis, o_xt):
    deg = degp_ref[0] + degp_ref[1] + 1.0
    dis = lax.rsqrt(deg)
    o_dis[0, 0] = dis
    o_xt[0, 0] = dis * x_ref[0, 0]


def _c0_call(x2, degp):
    return pl.pallas_call(
        _tc_c0,
        grid=(_GRID,),
        in_specs=[
            pl.BlockSpec((1, 1, _BLK), lambda i: (i, 0, 0)),
            pl.BlockSpec((2, _BLK), lambda i: (0, i)),
        ],
        out_specs=[pl.BlockSpec((1, 1, _BLK), lambda i: (i, 0, 0)),
                   pl.BlockSpec((1, 1, _BLK), lambda i: (i, 0, 0))],
        out_shape=[jax.ShapeDtypeStruct((_GRID, 1, _BLK), jnp.float32),
                   jax.ShapeDtypeStruct((_GRID, 1, _BLK), jnp.float32)],
    )(x2, degp)


def _tc_c1(p0p_ref, xt_ref, dis_ref, w1_ref, b1_ref, o_ht1):
    q = (p0p_ref[0] + p0p_ref[1] + xt_ref[0, 0]) * dis_ref[0, 0]
    h1 = jnp.maximum(q[:, None] * w1_ref[0][None, :] + b1_ref[0][None, :], 0.0)
    ht1 = dis_ref[0, 0][:, None] * h1
    o_ht1[0] = ht1[:, :32]
    o_ht1[1] = ht1[:, 32:]


def _c1_call(p0p, xt2, dis2, W1, b1):
    return pl.pallas_call(
        _tc_c1,
        grid=(_GRID,),
        in_specs=[
            pl.BlockSpec((2, _BLK), lambda i: (0, i)),
            pl.BlockSpec((1, 1, _BLK), lambda i: (i, 0, 0)),
            pl.BlockSpec((1, 1, _BLK), lambda i: (i, 0, 0)),
            pl.BlockSpec((1, 64), lambda i: (0, 0)),
            pl.BlockSpec((1, 64), lambda i: (0, 0)),
        ],
        out_specs=pl.BlockSpec((2, _BLK, 32), lambda i: (0, i, 0)),
        out_shape=jax.ShapeDtypeStruct((2, NROWS, 32), jnp.float32),
    )(p0p, xt2, dis2, W1, b1.reshape(1, 64))


def _tc_e(s1_ref, ht1_ref, dis_ref, w2_ref, b2_ref, o_ht2):
    dis = dis_ref[0, 0]
    q = jnp.concatenate([s1_ref[0] + ht1_ref[0], s1_ref[1] + ht1_ref[1]],
                        axis=1) * dis[:, None]
    h2 = jnp.dot(q, w2_ref[...], preferred_element_type=jnp.float32)
    h2 = jnp.maximum(h2 + b2_ref[0][None, :], 0.0)
    ht2 = dis[:, None] * h2
    for j in range(4):
        o_ht2[j] = ht2[:, j * 32:(j + 1) * 32]


def _e_call(s1, ht1, dis2, W2, b2):
    return pl.pallas_call(
        _tc_e,
        grid=(_GRID,),
        in_specs=[
            pl.BlockSpec((2, _BLK, 32), lambda i: (0, i, 0)),
            pl.BlockSpec((2, _BLK, 32), lambda i: (0, i, 0)),
            pl.BlockSpec((1, 1, _BLK), lambda i: (i, 0, 0)),
            pl.BlockSpec((64, 128), lambda i: (0, 0)),
            pl.BlockSpec((1, 128), lambda i: (0, 0)),
        ],
        out_specs=pl.BlockSpec((4, _BLK, 32), lambda i: (0, i, 0)),
        out_shape=jax.ShapeDtypeStruct((4, NROWS, 32), jnp.float32),
    )(s1, ht1, dis2, W2, b2.reshape(1, 128))


def _tc_g(s2_ref, ht2_ref, dis_ref, w3_ref, b3_ref, o_h3):
    dis = dis_ref[0, 0]
    q = jnp.concatenate([s2_ref[j] + ht2_ref[j] for j in range(4)],
                        axis=1) * dis[:, None]
    h3 = jnp.dot(q, w3_ref[...], preferred_element_type=jnp.float32)
    o_h3[...] = h3 + b3_ref[0][None, :]


def _g_call(s2, ht2, dis2, W3, b3):
    return pl.pallas_call(
        _tc_g,
        grid=(_GRID,),
        in_specs=[
            pl.BlockSpec((4, _BLK, 32), lambda i: (0, i, 0)),
            pl.BlockSpec((4, _BLK, 32), lambda i: (0, i, 0)),
            pl.BlockSpec((1, 1, _BLK), lambda i: (i, 0, 0)),
            pl.BlockSpec((128, 256), lambda i: (0, 0)),
            pl.BlockSpec((1, 256), lambda i: (0, 0)),
        ],
        out_specs=pl.BlockSpec((_BLK, 256), lambda i: (i, 0)),
        out_shape=jax.ShapeDtypeStruct((NROWS, 256), jnp.float32),
    )(s2, ht2, dis2, W3, b3.reshape(1, 256))


def _tc_head(psum_ref, pmax_ref, cntp_ref, fw1_ref, fb1_ref, fw2_ref, fb2_ref,
             o_ref):
    counts = cntp_ref[0, :G] + cntp_ref[1, :G]
    mean = psum_ref[:G] / jnp.maximum(counts, 1.0)[:, None]
    mx = jnp.where(counts[:, None] > 0, pmax_ref[:G], 0.0)
    z = jnp.concatenate([mean, mx], axis=1)
    z = jnp.dot(z, fw1_ref[...], preferred_element_type=jnp.float32)
    z = jnp.maximum(z + fb1_ref[0][None, :], 0.0)
    out = jnp.dot(z, fw2_ref[...], preferred_element_type=jnp.float32)
    o_ref[...] = out + fb2_ref[0][None, :]


def _head_call(psum, pmax, cntp, fW1, fb1, fW2, fb2):
    return pl.pallas_call(
        _tc_head,
        grid=(1,),
        in_specs=[
            pl.BlockSpec((GPAD, 256), lambda i: (0, 0)),
            pl.BlockSpec((GPAD, 256), lambda i: (0, 0)),
            pl.BlockSpec((2, GPAD), lambda i: (0, 0)),
            pl.BlockSpec((512, 128), lambda i: (0, 0)),
            pl.BlockSpec((1, 128), lambda i: (0, 0)),
            pl.BlockSpec((128, 12), lambda i: (0, 0)),
            pl.BlockSpec((1, 12), lambda i: (0, 0)),
        ],
        out_specs=pl.BlockSpec((G, 12), lambda i: (0, 0)),
        out_shape=jax.ShapeDtypeStruct((G, 12), jnp.float32),
    )(psum, pmax, cntp, fW1, fb1.reshape(1, 128), fW2, fb2.reshape(1, 12))


def kernel(x, edge_index, batch, W1, b1, W2, b2, W3, b3, fW1, fb1, fW2, fb2):
    src = edge_index[0]
    dst = edge_index[1]
    bat_pad = jnp.concatenate(
        [batch, jnp.full((NROWS - N,), G, jnp.int32)])
    x2 = jnp.pad(x[:, 0], (0, NROWS - N)).reshape(_GRID, 1, _BLK)

    degp, cntp = _deg_counts(dst, bat_pad)
    dis2, xt2 = _c0_call(x2, degp)
    src2 = src.reshape(ECH, 128)
    dst2 = dst.reshape(ECH, 128)
    (p0p,) = _p0_scatter(xt2.reshape(NROWS), src2, dst2)
    ht1 = _c1_call(p0p, xt2, dis2, W1, b1)
    (s1,) = _s1_scatter(ht1, src2, dst2)
    ht2 = _e_call(s1, ht1, dis2, W2, b2)
    (s2,) = _s2_scatter(ht2, src2, dst2)
    h3 = _g_call(s2, ht2, dis2, W3, b3)
    psum, pmax = _pool(h3.reshape(NROWS * 256), cntp)
    return _head_call(psum, pmax, cntp, fW1, fb1, fW2, fb2)


# speedup vs baseline: 28.6830x; 1.2389x over previous
import functools
import jax
import jax.numpy as jnp
from jax import lax
from jax.experimental import pallas as pl
from jax.experimental.pallas import tpu as pltpu
from jax.experimental.pallas import tpu_sc as plsc

N = 50000
E = 800000
G = 2000
NPAD = 51200
NROWS = 50176
GPAD = 2048
ECH = E // 128
BCH = NROWS // 128
NEG_INF = float("-inf")

_MESH = plsc.VectorSubcoreMesh(core_axis_name="c", subcore_axis_name="s")
_Z16 = functools.partial(jnp.zeros, (16,), jnp.float32)


def _fill(ref, n, value):
    v = jnp.full((16,), value, jnp.float32)

    def body(i, _):
        ref[pl.ds(i * 16, 16)] = v
        return 0

    lax.fori_loop(0, n // 16, body, 0)


def _fill2d(ref, rows, value):
    v = jnp.full((16,), value, jnp.float32)

    def body(i, _):
        r = i // 2
        col = (i % 2) * 16
        ref[r, pl.ds(col, 16)] = v
        return 0

    lax.fori_loop(0, rows * 2, body, 0)


def _sc_deg_counts(dst_hbm, bat_hbm, degp_hbm, cntp_hbm,
                   accd, accc, zbuf, obuf, idx):
    c = lax.axis_index("c")
    s = lax.axis_index("s")
    w = c * 16 + s
    _fill(zbuf, 3200, 0.0)
    _fill(obuf, 128, 1.0)
    pltpu.sync_copy(zbuf, accd.at[pl.ds(s * 3200, 3200)])
    pltpu.sync_copy(zbuf.at[pl.ds(0, 128)], accc.at[pl.ds(s * 128, 128)])
    plsc.subcore_barrier()

    def estep(k, _):
        pltpu.sync_copy(dst_hbm.at[pl.ds(k * 128, 128)], idx.at[0])
        pltpu.sync_copy(obuf, accd.at[idx.at[0]], add=True)
        return 0

    lax.fori_loop(w * ECH // 32, (w + 1) * ECH // 32, estep, 0)

    def bstep(k, _):
        pltpu.sync_copy(bat_hbm.at[pl.ds(k * 128, 128)], idx.at[0])
        pltpu.sync_copy(obuf, accc.at[idx.at[0]], add=True)
        return 0

    lax.fori_loop(w * BCH // 32, (w + 1) * BCH // 32, bstep, 0)

    plsc.subcore_barrier()
    pltpu.sync_copy(accd.at[pl.ds(s * 3200, 3200)],
                    degp_hbm.at[c].at[pl.ds(s * 3200, 3200)])
    pltpu.sync_copy(accc.at[pl.ds(s * 128, 128)],
                    cntp_hbm.at[c].at[pl.ds(s * 128, 128)])


_deg_counts = pl.kernel(
    _sc_deg_counts,
    out_type=[jax.ShapeDtypeStruct((2, NPAD), jnp.float32),
              jax.ShapeDtypeStruct((2, GPAD), jnp.float32)],
    mesh=_MESH,
    scratch_types=[
        pltpu.VMEM_SHARED((NPAD,), jnp.float32),
        pltpu.VMEM_SHARED((GPAD,), jnp.float32),
        pltpu.VMEM((3200,), jnp.float32),
        pltpu.VMEM((128,), jnp.float32),
        pltpu.VMEM((1, 128), jnp.int32),
    ],
)


def _sc_p0(xt_hbm, src2_hbm, dst2_hbm, p0p_hbm,
           acc, zbuf, vals0, vals1, vals2,
           idxs0, idxs1, idxs2, idxd0, idxd1, idxd2,
           sg0, sg1, sg2, ss0, ss1, ss2, st):
    c = lax.axis_index("c")
    s = lax.axis_index("s")
    w = c * 16 + s
    KB = 8
    NSB = 8
    MAIN = 32 * 192
    TAIL = ECH - MAIN
    _fill(zbuf, 3200, 0.0)
    pltpu.sync_copy(zbuf, acc.at[pl.ds(s * 3200, 3200)])
    plsc.subcore_barrier()
    vals = (vals0, vals1, vals2)
    idxs = (idxs0, idxs1, idxs2)
    idxd = (idxd0, idxd1, idxd2)
    sg = (sg0, sg1, sg2)
    ss = (ss0, ss1, ss2)

    def drain_scatters(q):
        for b in range(KB):
            pltpu.make_async_copy(vals[q].at[pl.ds(b * 128, 128)],
                                  acc.at[idxd[q].at[b]], ss[q]).wait()

    def drain_gathers(q):
        for b in range(KB):
            pltpu.make_async_copy(xt_hbm.at[idxs[q].at[b]],
                                  vals[q].at[pl.ds(b * 128, 128)], sg[q]).wait()

    def fire_scatters(q):
        for b in range(KB):
            pltpu.async_copy(vals[q].at[pl.ds(b * 128, 128)],
                             acc.at[idxd[q].at[b]], ss[q], add=True)

    def do_block(sb, q):
        qp = (q - 1) % 3
        row0 = w * 192 + (3 * sb + q) * KB
        @pl.when(sb > 0)
        def _():
            drain_scatters(q)
        pltpu.sync_copy(src2_hbm.at[pl.ds(row0, KB), :], idxs[q])
        pltpu.sync_copy(dst2_hbm.at[pl.ds(row0, KB), :], idxd[q])
        for b in range(KB):
            pltpu.async_copy(xt_hbm.at[idxs[q].at[b]],
                             vals[q].at[pl.ds(b * 128, 128)], sg[q])
        if q == 0:
            @pl.when(sb > 0)
            def _():
                drain_gathers(qp)
                fire_scatters(qp)
        else:
            drain_gathers(qp)
            fire_scatters(qp)

    def sbstep(sb, _):
        do_block(sb, 0)
        do_block(sb, 1)
        do_block(sb, 2)
        return 0

    lax.fori_loop(0, NSB, sbstep, 0)
    drain_gathers(2)
    fire_scatters(2)
    drain_scatters(0)
    drain_scatters(1)
    drain_scatters(2)

    def tstep(k, _):
        pltpu.sync_copy(src2_hbm.at[pl.ds(k, 1), :], idxs0.at[pl.ds(0, 1), :])
        pltpu.async_copy(xt_hbm.at[idxs0.at[0]],
                         vals0.at[pl.ds(0, 128)], st).wait()
        pltpu.sync_copy(dst2_hbm.at[pl.ds(k, 1), :], idxd0.at[pl.ds(0, 1), :])
        pltpu.sync_copy(vals0.at[pl.ds(0, 128)],
                        acc.at[idxd0.at[0]], add=True)
        return 0

    lax.fori_loop(MAIN + w * TAIL // 32, MAIN + (w + 1) * TAIL // 32, tstep, 0)

    plsc.subcore_barrier()
    pltpu.sync_copy(acc.at[pl.ds(s * 3200, 3200)],
                    p0p_hbm.at[c].at[pl.ds(s * 3200, 3200)])


_p0_scatter = pl.kernel(
    _sc_p0,
    out_type=[jax.ShapeDtypeStruct((2, NPAD), jnp.float32)],
    mesh=_MESH,
    compiler_params=pltpu.CompilerParams(use_tc_tiling_on_sc=False),
    scratch_types=[
        pltpu.VMEM_SHARED((NPAD,), jnp.float32),
        pltpu.VMEM((3200,), jnp.float32),
        pltpu.VMEM((1024,), jnp.float32),
        pltpu.VMEM((1024,), jnp.float32),
        pltpu.VMEM((1024,), jnp.float32),
        pltpu.VMEM((8, 128), jnp.int32),
        pltpu.VMEM((8, 128), jnp.int32),
        pltpu.VMEM((8, 128), jnp.int32),
        pltpu.VMEM((8, 128), jnp.int32),
        pltpu.VMEM((8, 128), jnp.int32),
        pltpu.VMEM((8, 128), jnp.int32),
        pltpu.SemaphoreType.DMA,
        pltpu.SemaphoreType.DMA,
        pltpu.SemaphoreType.DMA,
        pltpu.SemaphoreType.DMA,
        pltpu.SemaphoreType.DMA,
        pltpu.SemaphoreType.DMA,
        pltpu.SemaphoreType.DMA,
    ],
)


NPAD2 = 50048


def _make_row_scatter(num_chunks):
    per_core = num_chunks // 2
    KB = 2
    NSB = 32
    MAIN = 16 * 384
    TAIL = ECH - MAIN

    def body(ht_hbm, src2_hbm, dst2_hbm, out_hbm,
             acc, zrows, rows0, rows1, rows2,
             idxs0, idxs1, idxs2, idxs3, idxs4, idxs5,
             idxd0, idxd1, idxd2, idxd3, idxd4, idxd5,
             sg0, sg1, sg2, ss0, ss1, ss2,
             si0, si1, si2, si3, si4, si5, st):
        c = lax.axis_index("c")
        s = lax.axis_index("s")
        _fill2d(zrows, 32, 0.0)
        rows = (rows0, rows1, rows2)
        idxs = (idxs0, idxs1, idxs2, idxs3, idxs4, idxs5)
        idxd = (idxd0, idxd1, idxd2, idxd3, idxd4, idxd5)
        sg = (sg0, sg1, sg2)
        ss = (ss0, ss1, ss2)
        si = (si0, si1, si2, si3, si4, si5)

        def row0_of(sb, q):
            return s * 384 + (6 * sb + q) * KB

        def drain_scatters(r, qi):
            for b in range(KB):
                pltpu.make_async_copy(
                    rows[r].at[pl.ds(b * 128, 128), :],
                    acc.at[idxd[qi].at[b]], ss[r]).wait()

        def drain_gathers(r, qi):
            for b in range(KB):
                pltpu.make_async_copy(
                    ht_hbm.at[0].at[idxs[qi].at[b]],
                    rows[r].at[pl.ds(b * 128, 128), :], sg[r]).wait()

        def fire_scatters(r, qi):
            for b in range(KB):
                pltpu.async_copy(rows[r].at[pl.ds(b * 128, 128), :],
                                 acc.at[idxd[qi].at[b]], ss[r], add=True)

        def issue_idx(sb, q):
            r0 = row0_of(sb, q) if q < 6 else row0_of(sb + 1, q - 6)
            qi = q % 6
            pltpu.async_copy(src2_hbm.at[pl.ds(r0, KB), :], idxs[qi], si[qi])
            pltpu.async_copy(dst2_hbm.at[pl.ds(r0, KB), :], idxd[qi], si[qi])

        def wait_idx(qi):
            pltpu.make_async_copy(src2_hbm.at[pl.ds(0, KB), :],
                                  idxs[qi], si[qi]).wait()
            pltpu.make_async_copy(dst2_hbm.at[pl.ds(0, KB), :],
                                  idxd[qi], si[qi]).wait()

        for j in range(per_core):
            ch = c * per_core + j

            def zstep(i, _):
                pltpu.sync_copy(zrows, acc.at[pl.ds(s * 3128 + i * 32, 32), :])
                return 0

            lax.fori_loop(0, 97, zstep, 0)
            pltpu.sync_copy(zrows.at[pl.ds(0, 24), :],
                            acc.at[pl.ds(s * 3128 + 3104, 24), :])
            plsc.subcore_barrier()

            def fire_gathers(r, qi):
                for b in range(KB):
                    pltpu.async_copy(ht_hbm.at[ch].at[idxs[qi].at[b]],
                                     rows[r].at[pl.ds(b * 128, 128), :], sg[r])

            def do_block(sb, q):
                r = q % 3
                rp = (q - 1) % 3
                qp = (q - 1) % 6

                if q < 2:
                    @pl.when(sb > 0)
                    def _():
                        wait_idx(q)
                else:
                    wait_idx(q)

                if q < 3:
                    @pl.when(sb > 0)
                    def _():
                        drain_scatters(r, (q + 3) % 6)
                else:
                    drain_scatters(r, (q + 3) % 6)

                fire_gathers(r, q)

                if q < 4:
                    issue_idx(sb, q + 2)
                else:
                    @pl.when(sb < NSB - 1)
                    def _():
                        issue_idx(sb, q + 2)

                if q == 0:
                    @pl.when(sb > 0)
                    def _():
                        drain_gathers(rp, 5)
                        fire_scatters(rp, 5)
                else:
                    drain_gathers(rp, qp)
                    fire_scatters(rp, qp)

            def sbstep(sb, _):
                for q in range(6):
                    do_block(sb, q)
                return 0

            pltpu.sync_copy(src2_hbm.at[pl.ds(row0_of(0, 0), KB), :], idxs0)
            pltpu.sync_copy(dst2_hbm.at[pl.ds(row0_of(0, 0), KB), :], idxd0)
            pltpu.sync_copy(src2_hbm.at[pl.ds(row0_of(0, 1), KB), :], idxs1)
            pltpu.sync_copy(dst2_hbm.at[pl.ds(row0_of(0, 1), KB), :], idxd1)
            lax.fori_loop(0, NSB, sbstep, 0)
            drain_gathers(2, 5)
            fire_scatters(2, 5)
            drain_scatters(0, 3)
            drain_scatters(1, 4)
            drain_scatters(2, 5)

            def tstep(k, _):
                pltpu.sync_copy(src2_hbm.at[pl.ds(k, 1), :],
                                idxs0.at[pl.ds(0, 1), :])
                pltpu.async_copy(ht_hbm.at[ch].at[idxs0.at[0]],
                                 rows0.at[pl.ds(0, 128), :], st).wait()
                pltpu.sync_copy(dst2_hbm.at[pl.ds(k, 1), :],
                                idxd0.at[pl.ds(0, 1), :])
                pltpu.sync_copy(rows0.at[pl.ds(0, 128), :],
                                acc.at[idxd0.at[0]], add=True)
                return 0

            lax.fori_loop(MAIN + s * TAIL // 16, MAIN + (s + 1) * TAIL // 16,
                          tstep, 0)

            plsc.subcore_barrier()
            pltpu.sync_copy(acc.at[pl.ds(s * 3128, 3128), :],
                            out_hbm.at[ch].at[pl.ds(s * 3128, 3128), :])
            plsc.subcore_barrier()

    return pl.kernel(
        body,
        out_type=[jax.ShapeDtypeStruct((num_chunks, NPAD2, 32), jnp.float32)],
        mesh=_MESH,
        compiler_params=pltpu.CompilerParams(use_tc_tiling_on_sc=False),
        scratch_types=[
            pltpu.VMEM_SHARED((NPAD2, 32), jnp.float32),
            pltpu.VMEM((32, 32), jnp.float32),
            pltpu.VMEM((256, 32), jnp.float32),
            pltpu.VMEM((256, 32), jnp.float32),
            pltpu.VMEM((256, 32), jnp.float32),
        ] + [pltpu.VMEM((2, 128), jnp.int32)] * 12
          + [pltpu.SemaphoreType.DMA] * 13,
    )


_s1_scatter = _make_row_scatter(2)
_s2_scatter = _make_row_scatter(4)


def _sc_pool(h3_hbm, cntp_hbm, psum_hbm, pmax_hbm,
             cbuf, cbuf2, rbuf, osum, omax):
    c = lax.axis_index("c")
    s = lax.axis_index("s")
    wid = c * 16 + s
    pltpu.sync_copy(cntp_hbm.at[0], cbuf.at[pl.ds(0, GPAD)])
    pltpu.sync_copy(cntp_hbm.at[1], cbuf2)

    def addc(i, _):
        sl = pl.ds(i * 16, 16)
        cbuf[sl] = cbuf[sl] + cbuf2[sl]
        return 0

    lax.fori_loop(0, GPAD // 16, addc, 0)

    def pre(i, acc16):
        return acc16 + cbuf[pl.ds(i * 16, 16)]

    acc16 = lax.fori_loop(0, wid * 4, pre, _Z16())
    start0 = jnp.sum(acc16).astype(jnp.int32)

    def graph_step(g_local, start):
        g = wid * 64 + g_local
        cnt = cbuf[pl.ds(g, 16)][0].astype(jnp.int32)
        nch = (cnt + 31) // 32

        def chunk_step(k, carry):
            rowstart = jnp.minimum(start + k * 32, NROWS - 32)
            pltpu.sync_copy(h3_hbm.at[pl.ds(rowstart * 256, 32 * 256)], rbuf)
            m = jnp.minimum(32, cnt - k * 32)

            def row_step(r, carry2):
                sums, maxs = carry2
                new_s = []
                new_m = []
                for jj in range(16):
                    v = rbuf[pl.ds(r * 256 + jj * 16, 16)]
                    new_s.append(sums[jj] + v)
                    new_m.append(jnp.maximum(maxs[jj], v))
                return tuple(new_s), tuple(new_m)

            return lax.fori_loop(0, m, row_step, carry)

        init = (tuple(_Z16() for _ in range(16)),
                tuple(jnp.full((16,), NEG_INF, jnp.float32) for _ in range(16)))
        sums, maxs = lax.fori_loop(0, nch, chunk_step, init)
        for jj in range(16):
            osum[g_local, pl.ds(jj * 16, 16)] = sums[jj]
            omax[g_local, pl.ds(jj * 16, 16)] = maxs[jj]
        return start + cnt

    lax.fori_loop(0, 64, graph_step, start0)
    pltpu.sync_copy(osum, psum_hbm.at[pl.ds(wid * 64, 64), :])
    pltpu.sync_copy(omax, pmax_hbm.at[pl.ds(wid * 64, 64), :])


_pool = pl.kernel(
    _sc_pool,
    out_type=[jax.ShapeDtypeStruct((GPAD, 256), jnp.float32),
              jax.ShapeDtypeStruct((GPAD, 256), jnp.float32)],
    mesh=_MESH,
    compiler_params=pltpu.CompilerParams(needs_layout_passes=False),
    scratch_types=[
        pltpu.VMEM((GPAD + 16,), jnp.float32),
        pltpu.VMEM((GPAD,), jnp.float32),
        pltpu.VMEM((32 * 256,), jnp.float32),
        pltpu.VMEM((64, 256), jnp.float32),
        pltpu.VMEM((64, 256), jnp.float32),
    ],
)


_BLK = 1024
_GRID = NROWS // _BLK


def _tc_c0(x_ref, degp_ref, o_dis, o_xt):
    deg = degp_ref[0] + degp_ref[1] + 1.0
    dis = lax.rsqrt(deg)
    o_dis[0, 0] = dis
    o_xt[0, 0] = dis * x_ref[0, 0]


def _c0_call(x2, degp):
    return pl.pallas_call(
        _tc_c0,
        grid=(_GRID,),
        in_specs=[
            pl.BlockSpec((1, 1, _BLK), lambda i: (i, 0, 0)),
            pl.BlockSpec((2, _BLK), lambda i: (0, i)),
        ],
        out_specs=[pl.BlockSpec((1, 1, _BLK), lambda i: (i, 0, 0)),
                   pl.BlockSpec((1, 1, _BLK), lambda i: (i, 0, 0))],
        out_shape=[jax.ShapeDtypeStruct((_GRID, 1, _BLK), jnp.float32),
                   jax.ShapeDtypeStruct((_GRID, 1, _BLK), jnp.float32)],
    )(x2, degp)


def _tc_c1(p0p_ref, xt_ref, dis_ref, w1_ref, b1_ref, o_ht1):
    q = (p0p_ref[0] + p0p_ref[1] + xt_ref[0, 0]) * dis_ref[0, 0]
    h1 = jnp.maximum(q[:, None] * w1_ref[0][None, :] + b1_ref[0][None, :], 0.0)
    ht1 = dis_ref[0, 0][:, None] * h1
    o_ht1[0] = ht1[:, :32]
    o_ht1[1] = ht1[:, 32:]


def _c1_call(p0p, xt2, dis2, W1, b1):
    return pl.pallas_call(
        _tc_c1,
        grid=(_GRID,),
        in_specs=[
            pl.BlockSpec((2, _BLK), lambda i: (0, i)),
            pl.BlockSpec((1, 1, _BLK), lambda i: (i, 0, 0)),
            pl.BlockSpec((1, 1, _BLK), lambda i: (i, 0, 0)),
            pl.BlockSpec((1, 64), lambda i: (0, 0)),
            pl.BlockSpec((1, 64), lambda i: (0, 0)),
        ],
        out_specs=pl.BlockSpec((2, _BLK, 32), lambda i: (0, i, 0)),
        out_shape=jax.ShapeDtypeStruct((2, NROWS, 32), jnp.float32),
    )(p0p, xt2, dis2, W1, b1.reshape(1, 64))


def _tc_e(s1_ref, ht1_ref, dis_ref, w2_ref, b2_ref, o_ht2):
    dis = dis_ref[0, 0]
    q = jnp.concatenate([s1_ref[0] + ht1_ref[0], s1_ref[1] + ht1_ref[1]],
                        axis=1) * dis[:, None]
    h2 = jnp.dot(q, w2_ref[...], preferred_element_type=jnp.float32)
    h2 = jnp.maximum(h2 + b2_ref[0][None, :], 0.0)
    ht2 = dis[:, None] * h2
    for j in range(4):
        o_ht2[j] = ht2[:, j * 32:(j + 1) * 32]


def _e_call(s1, ht1, dis2, W2, b2):
    return pl.pallas_call(
        _tc_e,
        grid=(_GRID,),
        in_specs=[
            pl.BlockSpec((2, _BLK, 32), lambda i: (0, i, 0)),
            pl.BlockSpec((2, _BLK, 32), lambda i: (0, i, 0)),
            pl.BlockSpec((1, 1, _BLK), lambda i: (i, 0, 0)),
            pl.BlockSpec((64, 128), lambda i: (0, 0)),
            pl.BlockSpec((1, 128), lambda i: (0, 0)),
        ],
        out_specs=pl.BlockSpec((4, _BLK, 32), lambda i: (0, i, 0)),
        out_shape=jax.ShapeDtypeStruct((4, NROWS, 32), jnp.float32),
    )(s1, ht1, dis2, W2, b2.reshape(1, 128))


def _tc_g(s2_ref, ht2_ref, dis_ref, w3_ref, b3_ref, o_h3):
    dis = dis_ref[0, 0]
    q = jnp.concatenate([s2_ref[j] + ht2_ref[j] for j in range(4)],
                        axis=1) * dis[:, None]
    h3 = jnp.dot(q, w3_ref[...], preferred_element_type=jnp.float32)
    o_h3[...] = h3 + b3_ref[0][None, :]


def _g_call(s2, ht2, dis2, W3, b3):
    return pl.pallas_call(
        _tc_g,
        grid=(_GRID,),
        in_specs=[
            pl.BlockSpec((4, _BLK, 32), lambda i: (0, i, 0)),
            pl.BlockSpec((4, _BLK, 32), lambda i: (0, i, 0)),
            pl.BlockSpec((1, 1, _BLK), lambda i: (i, 0, 0)),
            pl.BlockSpec((128, 256), lambda i: (0, 0)),
            pl.BlockSpec((1, 256), lambda i: (0, 0)),
        ],
        out_specs=pl.BlockSpec((_BLK, 256), lambda i: (i, 0)),
        out_shape=jax.ShapeDtypeStruct((NROWS, 256), jnp.float32),
    )(s2, ht2, dis2, W3, b3.reshape(1, 256))


def _tc_head(psum_ref, pmax_ref, cntp_ref, fw1_ref, fb1_ref, fw2_ref, fb2_ref,
             o_ref):
    counts = cntp_ref[0, :G] + cntp_ref[1, :G]
    mean = psum_ref[:G] / jnp.maximum(counts, 1.0)[:, None]
    mx = jnp.where(counts[:, None] > 0, pmax_ref[:G], 0.0)
    z = jnp.concatenate([mean, mx], axis=1)
    z = jnp.dot(z, fw1_ref[...], preferred_element_type=jnp.float32)
    z = jnp.maximum(z + fb1_ref[0][None, :], 0.0)
    out = jnp.dot(z, fw2_ref[...], preferred_element_type=jnp.float32)
    o_ref[...] = out + fb2_ref[0][None, :]


def _head_call(psum, pmax, cntp, fW1, fb1, fW2, fb2):
    return pl.pallas_call(
        _tc_head,
        grid=(1,),
        in_specs=[
            pl.BlockSpec((GPAD, 256), lambda i: (0, 0)),
            pl.BlockSpec((GPAD, 256), lambda i: (0, 0)),
            pl.BlockSpec((2, GPAD), lambda i: (0, 0)),
            pl.BlockSpec((512, 128), lambda i: (0, 0)),
            pl.BlockSpec((1, 128), lambda i: (0, 0)),
            pl.BlockSpec((128, 12), lambda i: (0, 0)),
            pl.BlockSpec((1, 12), lambda i: (0, 0)),
        ],
        out_specs=pl.BlockSpec((G, 12), lambda i: (0, 0)),
        out_shape=jax.ShapeDtypeStruct((G, 12), jnp.float32),
    )(psum, pmax, cntp, fW1, fb1.reshape(1, 128), fW2, fb2.reshape(1, 12))


def kernel(x, edge_index, batch, W1, b1, W2, b2, W3, b3, fW1, fb1, fW2, fb2):
    src = edge_index[0]
    dst = edge_index[1]
    bat_pad = jnp.concatenate(
        [batch, jnp.full((NROWS - N,), G, jnp.int32)])
    x2 = jnp.pad(x[:, 0], (0, NROWS - N)).reshape(_GRID, 1, _BLK)

    degp, cntp = _deg_counts(dst, bat_pad)
    dis2, xt2 = _c0_call(x2, degp)
    src2 = src.reshape(ECH, 128)
    dst2 = dst.reshape(ECH, 128)
    (p0p,) = _p0_scatter(xt2.reshape(NROWS), src2, dst2)
    ht1 = _c1_call(p0p, xt2, dis2, W1, b1)
    (s1,) = _s1_scatter(ht1, src2, dst2)
    ht2 = _e_call(s1, ht1, dis2, W2, b2)
    (s2,) = _s2_scatter(ht2, src2, dst2)
    h3 = _g_call(s2, ht2, dis2, W3, b3)
    psum, pmax = _pool(h3.reshape(NROWS * 256), cntp)
    return _head_call(psum, pmax, cntp, fW1, fb1, fW2, fb2)


# speedup vs baseline: 31.3969x; 1.0946x over previous
import functools
import jax
import jax.numpy as jnp
from jax import lax
from jax.experimental import pallas as pl
from jax.experimental.pallas import tpu as pltpu
from jax.experimental.pallas import tpu_sc as plsc

N = 50000
E = 800000
G = 2000
NPAD = 51200
NROWS = 50176
GPAD = 2048
ECH = E // 128
BCH = NROWS // 128
NEG_INF = float("-inf")

_MESH = plsc.VectorSubcoreMesh(core_axis_name="c", subcore_axis_name="s")
_Z16 = functools.partial(jnp.zeros, (16,), jnp.float32)


def _fill(ref, n, value):
    v = jnp.full((16,), value, jnp.float32)

    def body(i, _):
        ref[pl.ds(i * 16, 16)] = v
        return 0

    lax.fori_loop(0, n // 16, body, 0)


def _fill2d(ref, rows, value):
    v = jnp.full((16,), value, jnp.float32)

    def body(i, _):
        r = i // 2
        col = (i % 2) * 16
        ref[r, pl.ds(col, 16)] = v
        return 0

    lax.fori_loop(0, rows * 2, body, 0)


def _sc_deg_counts(dst2_hbm, bat2_hbm, degp_hbm, cntp_hbm,
                   accd, accc, zbuf, obuf, idxd0, idxd1, idxd2,
                   ss0, ss1, ss2):
    c = lax.axis_index("c")
    s = lax.axis_index("s")
    w = c * 16 + s
    KB = 8
    NSB = 8
    MAIN = 32 * 192
    TAIL = ECH - MAIN
    _fill(zbuf, 3200, 0.0)
    _fill(obuf, 1024, 1.0)
    pltpu.sync_copy(zbuf, accd.at[pl.ds(s * 3200, 3200)])
    pltpu.sync_copy(zbuf.at[pl.ds(0, 128)], accc.at[pl.ds(s * 128, 128)])
    plsc.subcore_barrier()
    idxd = (idxd0, idxd1, idxd2)
    ss = (ss0, ss1, ss2)

    def drain_scatters(q):
        for b in range(KB):
            pltpu.make_async_copy(obuf.at[pl.ds(b * 128, 128)],
                                  accd.at[idxd[q].at[b]], ss[q]).wait()

    def do_block(sb, q):
        row0 = w * 192 + (3 * sb + q) * KB

        @pl.when(sb > 0)
        def _():
            drain_scatters(q)

        pltpu.sync_copy(dst2_hbm.at[pl.ds(row0, KB), :], idxd[q])
        for b in range(KB):
            pltpu.async_copy(obuf.at[pl.ds(b * 128, 128)],
                             accd.at[idxd[q].at[b]], ss[q], add=True)

    def sbstep(sb, _):
        do_block(sb, 0)
        do_block(sb, 1)
        do_block(sb, 2)
        return 0

    lax.fori_loop(0, NSB, sbstep, 0)
    drain_scatters(0)
    drain_scatters(1)
    drain_scatters(2)

    def tstep(k, _):
        pltpu.sync_copy(dst2_hbm.at[pl.ds(k, 1), :], idxd0.at[pl.ds(0, 1), :])
        pltpu.sync_copy(obuf.at[pl.ds(0, 128)], accd.at[idxd0.at[0]], add=True)
        return 0

    lax.fori_loop(MAIN + w * TAIL // 32, MAIN + (w + 1) * TAIL // 32, tstep, 0)

    def bstep(k, _):
        pltpu.sync_copy(bat2_hbm.at[pl.ds(k, 1), :], idxd0.at[pl.ds(0, 1), :])
        pltpu.sync_copy(obuf.at[pl.ds(0, 128)], accc.at[idxd0.at[0]], add=True)
        return 0

    lax.fori_loop(w * BCH // 32, (w + 1) * BCH // 32, bstep, 0)

    plsc.subcore_barrier()
    pltpu.sync_copy(accd.at[pl.ds(s * 3200, 3200)],
                    degp_hbm.at[c].at[pl.ds(s * 3200, 3200)])
    pltpu.sync_copy(accc.at[pl.ds(s * 128, 128)],
                    cntp_hbm.at[c].at[pl.ds(s * 128, 128)])


_deg_counts = pl.kernel(
    _sc_deg_counts,
    out_type=[jax.ShapeDtypeStruct((2, NPAD), jnp.float32),
              jax.ShapeDtypeStruct((2, GPAD), jnp.float32)],
    mesh=_MESH,
    compiler_params=pltpu.CompilerParams(use_tc_tiling_on_sc=False),
    scratch_types=[
        pltpu.VMEM_SHARED((NPAD,), jnp.float32),
        pltpu.VMEM_SHARED((GPAD,), jnp.float32),
        pltpu.VMEM((3200,), jnp.float32),
        pltpu.VMEM((1024,), jnp.float32),
        pltpu.VMEM((8, 128), jnp.int32),
        pltpu.VMEM((8, 128), jnp.int32),
        pltpu.VMEM((8, 128), jnp.int32),
        pltpu.SemaphoreType.DMA,
        pltpu.SemaphoreType.DMA,
        pltpu.SemaphoreType.DMA,
    ],
)


def _sc_p0(xt_hbm, src2_hbm, dst2_hbm, p0p_hbm,
           acc, zbuf, vals0, vals1, vals2,
           idxs0, idxs1, idxs2, idxd0, idxd1, idxd2,
           sg0, sg1, sg2, ss0, ss1, ss2, st):
    c = lax.axis_index("c")
    s = lax.axis_index("s")
    w = c * 16 + s
    KB = 8
    NSB = 8
    MAIN = 32 * 192
    TAIL = ECH - MAIN
    _fill(zbuf, 3200, 0.0)
    pltpu.sync_copy(zbuf, acc.at[pl.ds(s * 3200, 3200)])
    plsc.subcore_barrier()
    vals = (vals0, vals1, vals2)
    idxs = (idxs0, idxs1, idxs2)
    idxd = (idxd0, idxd1, idxd2)
    sg = (sg0, sg1, sg2)
    ss = (ss0, ss1, ss2)

    def drain_scatters(q):
        for b in range(KB):
            pltpu.make_async_copy(vals[q].at[pl.ds(b * 128, 128)],
                                  acc.at[idxd[q].at[b]], ss[q]).wait()

    def drain_gathers(q):
        for b in range(KB):
            pltpu.make_async_copy(xt_hbm.at[idxs[q].at[b]],
                                  vals[q].at[pl.ds(b * 128, 128)], sg[q]).wait()

    def fire_scatters(q):
        for b in range(KB):
            pltpu.async_copy(vals[q].at[pl.ds(b * 128, 128)],
                             acc.at[idxd[q].at[b]], ss[q], add=True)

    def do_block(sb, q):
        qp = (q - 1) % 3
        row0 = w * 192 + (3 * sb + q) * KB
        @pl.when(sb > 0)
        def _():
            drain_scatters(q)
        pltpu.sync_copy(src2_hbm.at[pl.ds(row0, KB), :], idxs[q])
        pltpu.sync_copy(dst2_hbm.at[pl.ds(row0, KB), :], idxd[q])
        for b in range(KB):
            pltpu.async_copy(xt_hbm.at[idxs[q].at[b]],
                             vals[q].at[pl.ds(b * 128, 128)], sg[q])
        if q == 0:
            @pl.when(sb > 0)
            def _():
                drain_gathers(qp)
                fire_scatters(qp)
        else:
            drain_gathers(qp)
            fire_scatters(qp)

    def sbstep(sb, _):
        do_block(sb, 0)
        do_block(sb, 1)
        do_block(sb, 2)
        return 0

    lax.fori_loop(0, NSB, sbstep, 0)
    drain_gathers(2)
    fire_scatters(2)
    drain_scatters(0)
    drain_scatters(1)
    drain_scatters(2)

    def tstep(k, _):
        pltpu.sync_copy(src2_hbm.at[pl.ds(k, 1), :], idxs0.at[pl.ds(0, 1), :])
        pltpu.async_copy(xt_hbm.at[idxs0.at[0]],
                         vals0.at[pl.ds(0, 128)], st).wait()
        pltpu.sync_copy(dst2_hbm.at[pl.ds(k, 1), :], idxd0.at[pl.ds(0, 1), :])
        pltpu.sync_copy(vals0.at[pl.ds(0, 128)],
                        acc.at[idxd0.at[0]], add=True)
        return 0

    lax.fori_loop(MAIN + w * TAIL // 32, MAIN + (w + 1) * TAIL // 32, tstep, 0)

    plsc.subcore_barrier()
    pltpu.sync_copy(acc.at[pl.ds(s * 3200, 3200)],
                    p0p_hbm.at[c].at[pl.ds(s * 3200, 3200)])


_p0_scatter = pl.kernel(
    _sc_p0,
    out_type=[jax.ShapeDtypeStruct((2, NPAD), jnp.float32)],
    mesh=_MESH,
    compiler_params=pltpu.CompilerParams(use_tc_tiling_on_sc=False),
    scratch_types=[
        pltpu.VMEM_SHARED((NPAD,), jnp.float32),
        pltpu.VMEM((3200,), jnp.float32),
        pltpu.VMEM((1024,), jnp.float32),
        pltpu.VMEM((1024,), jnp.float32),
        pltpu.VMEM((1024,), jnp.float32),
        pltpu.VMEM((8, 128), jnp.int32),
        pltpu.VMEM((8, 128), jnp.int32),
        pltpu.VMEM((8, 128), jnp.int32),
        pltpu.VMEM((8, 128), jnp.int32),
        pltpu.VMEM((8, 128), jnp.int32),
        pltpu.VMEM((8, 128), jnp.int32),
        pltpu.SemaphoreType.DMA,
        pltpu.SemaphoreType.DMA,
        pltpu.SemaphoreType.DMA,
        pltpu.SemaphoreType.DMA,
        pltpu.SemaphoreType.DMA,
        pltpu.SemaphoreType.DMA,
        pltpu.SemaphoreType.DMA,
    ],
)


NPAD2 = 50048


def _make_row_scatter(num_chunks):
    per_core = num_chunks // 2
    KB = 2
    NSB = 32
    MAIN = 16 * 384
    TAIL = ECH - MAIN

    def body(ht_hbm, src2_hbm, dst2_hbm, out_hbm,
             acc, zrows, rows0, rows1, rows2,
             idxs0, idxs1, idxs2, idxs3, idxs4, idxs5,
             idxd0, idxd1, idxd2, idxd3, idxd4, idxd5,
             sg0, sg1, sg2, ss0, ss1, ss2,
             si0, si1, si2, si3, si4, si5, st):
        c = lax.axis_index("c")
        s = lax.axis_index("s")
        _fill2d(zrows, 32, 0.0)
        rows = (rows0, rows1, rows2)
        idxs = (idxs0, idxs1, idxs2, idxs3, idxs4, idxs5)
        idxd = (idxd0, idxd1, idxd2, idxd3, idxd4, idxd5)
        sg = (sg0, sg1, sg2)
        ss = (ss0, ss1, ss2)
        si = (si0, si1, si2, si3, si4, si5)

        def row0_of(sb, q):
            return s * 384 + (6 * sb + q) * KB

        def drain_scatters(r, qi):
            for b in range(KB):
                pltpu.make_async_copy(
                    rows[r].at[pl.ds(b * 128, 128), :],
                    acc.at[idxd[qi].at[b]], ss[r]).wait()

        def drain_gathers(r, qi):
            for b in range(KB):
                pltpu.make_async_copy(
                    ht_hbm.at[0].at[idxs[qi].at[b]],
                    rows[r].at[pl.ds(b * 128, 128), :], sg[r]).wait()

        def fire_scatters(r, qi):
            for b in range(KB):
                pltpu.async_copy(rows[r].at[pl.ds(b * 128, 128), :],
                                 acc.at[idxd[qi].at[b]], ss[r], add=True)

        def issue_idx(sb, q):
            r0 = row0_of(sb, q) if q < 6 else row0_of(sb + 1, q - 6)
            qi = q % 6
            pltpu.async_copy(src2_hbm.at[pl.ds(r0, KB), :], idxs[qi], si[qi])
            pltpu.async_copy(dst2_hbm.at[pl.ds(r0, KB), :], idxd[qi], si[qi])

        def wait_idx(qi):
            pltpu.make_async_copy(src2_hbm.at[pl.ds(0, KB), :],
                                  idxs[qi], si[qi]).wait()
            pltpu.make_async_copy(dst2_hbm.at[pl.ds(0, KB), :],
                                  idxd[qi], si[qi]).wait()

        for j in range(per_core):
            ch = c * per_core + j

            def zstep(i, _):
                pltpu.async_copy(zrows, acc.at[pl.ds(s * 3128 + i * 32, 32), :],
                                 st)
                return 0

            lax.fori_loop(0, 97, zstep, 0)
            pltpu.sync_copy(zrows.at[pl.ds(0, 24), :],
                            acc.at[pl.ds(s * 3128 + 3104, 24), :])

            def zwait(i, _):
                pltpu.make_async_copy(
                    zrows, acc.at[pl.ds(s * 3128, 32), :], st).wait()
                return 0

            lax.fori_loop(0, 97, zwait, 0)
            plsc.subcore_barrier()

            def fire_gathers(r, qi):
                for b in range(KB):
                    pltpu.async_copy(ht_hbm.at[ch].at[idxs[qi].at[b]],
                                     rows[r].at[pl.ds(b * 128, 128), :], sg[r])

            def do_block(sb, q):
                r = q % 3
                rp = (q - 1) % 3
                qp = (q - 1) % 6

                if q < 2:
                    @pl.when(sb > 0)
                    def _():
                        wait_idx(q)
                else:
                    wait_idx(q)

                if q < 3:
                    @pl.when(sb > 0)
                    def _():
                        drain_scatters(r, (q + 3) % 6)
                else:
                    drain_scatters(r, (q + 3) % 6)

                fire_gathers(r, q)

                if q < 4:
                    issue_idx(sb, q + 2)
                else:
                    @pl.when(sb < NSB - 1)
                    def _():
                        issue_idx(sb, q + 2)

                if q == 0:
                    @pl.when(sb > 0)
                    def _():
                        drain_gathers(rp, 5)
                        fire_scatters(rp, 5)
                else:
                    drain_gathers(rp, qp)
                    fire_scatters(rp, qp)

            def sbstep(sb, _):
                for q in range(6):
                    do_block(sb, q)
                return 0

            pltpu.sync_copy(src2_hbm.at[pl.ds(row0_of(0, 0), KB), :], idxs0)
            pltpu.sync_copy(dst2_hbm.at[pl.ds(row0_of(0, 0), KB), :], idxd0)
            pltpu.sync_copy(src2_hbm.at[pl.ds(row0_of(0, 1), KB), :], idxs1)
            pltpu.sync_copy(dst2_hbm.at[pl.ds(row0_of(0, 1), KB), :], idxd1)
            lax.fori_loop(0, NSB, sbstep, 0)
            drain_gathers(2, 5)
            fire_scatters(2, 5)
            drain_scatters(0, 3)
            drain_scatters(1, 4)
            drain_scatters(2, 5)

            def tstep(k, _):
                pltpu.sync_copy(src2_hbm.at[pl.ds(k, 1), :],
                                idxs0.at[pl.ds(0, 1), :])
                pltpu.async_copy(ht_hbm.at[ch].at[idxs0.at[0]],
                                 rows0.at[pl.ds(0, 128), :], st).wait()
                pltpu.sync_copy(dst2_hbm.at[pl.ds(k, 1), :],
                                idxd0.at[pl.ds(0, 1), :])
                pltpu.sync_copy(rows0.at[pl.ds(0, 128), :],
                                acc.at[idxd0.at[0]], add=True)
                return 0

            lax.fori_loop(MAIN + s * TAIL // 16, MAIN + (s + 1) * TAIL // 16,
                          tstep, 0)

            plsc.subcore_barrier()
            pltpu.sync_copy(acc.at[pl.ds(s * 3128, 3128), :],
                            out_hbm.at[ch].at[pl.ds(s * 3128, 3128), :])
            plsc.subcore_barrier()

    return pl.kernel(
        body,
        out_type=[jax.ShapeDtypeStruct((num_chunks, NPAD2, 32), jnp.float32)],
        mesh=_MESH,
        compiler_params=pltpu.CompilerParams(use_tc_tiling_on_sc=False),
        scratch_types=[
            pltpu.VMEM_SHARED((NPAD2, 32), jnp.float32),
            pltpu.VMEM((32, 32), jnp.float32),
            pltpu.VMEM((256, 32), jnp.float32),
            pltpu.VMEM((256, 32), jnp.float32),
            pltpu.VMEM((256, 32), jnp.float32),
        ] + [pltpu.VMEM((2, 128), jnp.int32)] * 12
          + [pltpu.SemaphoreType.DMA] * 13,
    )


_s1_scatter = _make_row_scatter(2)
_s2_scatter = _make_row_scatter(4)


def _sc_pool(h3_hbm, cntp_hbm, psum_hbm, pmax_hbm,
             cbuf, cbuf2, rbuf, osum, omax):
    c = lax.axis_index("c")
    s = lax.axis_index("s")
    wid = c * 16 + s
    pltpu.sync_copy(cntp_hbm.at[0], cbuf.at[pl.ds(0, GPAD)])
    pltpu.sync_copy(cntp_hbm.at[1], cbuf2)

    def addc(i, _):
        sl = pl.ds(i * 16, 16)
        cbuf[sl] = cbuf[sl] + cbuf2[sl]
        return 0

    lax.fori_loop(0, GPAD // 16, addc, 0)

    def pre(i, acc16):
        return acc16 + cbuf[pl.ds(i * 16, 16)]

    acc16 = lax.fori_loop(0, wid * 4, pre, _Z16())
    start0 = jnp.sum(acc16).astype(jnp.int32)

    def graph_step(g_local, start):
        g = wid * 64 + g_local
        cnt = cbuf[pl.ds(g, 16)][0].astype(jnp.int32)
        nch = (cnt + 31) // 32

        def chunk_step(k, carry):
            rowstart = jnp.minimum(start + k * 32, NROWS - 32)
            pltpu.sync_copy(h3_hbm.at[pl.ds(rowstart * 256, 32 * 256)], rbuf)
            m = jnp.minimum(32, cnt - k * 32)

            def row_step(r, carry2):
                sums, maxs = carry2
                new_s = []
                new_m = []
                for jj in range(16):
                    v = rbuf[pl.ds(r * 256 + jj * 16, 16)]
                    new_s.append(sums[jj] + v)
                    new_m.append(jnp.maximum(maxs[jj], v))
                return tuple(new_s), tuple(new_m)

            return lax.fori_loop(0, m, row_step, carry)

        init = (tuple(_Z16() for _ in range(16)),
                tuple(jnp.full((16,), NEG_INF, jnp.float32) for _ in range(16)))
        sums, maxs = lax.fori_loop(0, nch, chunk_step, init)
        for jj in range(16):
            osum[g_local, pl.ds(jj * 16, 16)] = sums[jj]
            omax[g_local, pl.ds(jj * 16, 16)] = maxs[jj]
        return start + cnt

    lax.fori_loop(0, 64, graph_step, start0)
    pltpu.sync_copy(osum, psum_hbm.at[pl.ds(wid * 64, 64), :])
    pltpu.sync_copy(omax, pmax_hbm.at[pl.ds(wid * 64, 64), :])


_pool = pl.kernel(
    _sc_pool,
    out_type=[jax.ShapeDtypeStruct((GPAD, 256), jnp.float32),
              jax.ShapeDtypeStruct((GPAD, 256), jnp.float32)],
    mesh=_MESH,
    compiler_params=pltpu.CompilerParams(needs_layout_passes=False),
    scratch_types=[
        pltpu.VMEM((GPAD + 16,), jnp.float32),
        pltpu.VMEM((GPAD,), jnp.float32),
        pltpu.VMEM((32 * 256,), jnp.float32),
        pltpu.VMEM((64, 256), jnp.float32),
        pltpu.VMEM((64, 256), jnp.float32),
    ],
)


_BLK = 1024
_GRID = NROWS // _BLK


def _tc_c0(x_ref, degp_ref, o_dis, o_xt):
    deg = degp_ref[0] + degp_ref[1] + 1.0
    dis = lax.rsqrt(deg)
    o_dis[0, 0] = dis
    o_xt[0, 0] = dis * x_ref[0, 0]


def _c0_call(x2, degp):
    return pl.pallas_call(
        _tc_c0,
        grid=(_GRID,),
        in_specs=[
            pl.BlockSpec((1, 1, _BLK), lambda i: (i, 0, 0)),
            pl.BlockSpec((2, _BLK), lambda i: (0, i)),
        ],
        out_specs=[pl.BlockSpec((1, 1, _BLK), lambda i: (i, 0, 0)),
                   pl.BlockSpec((1, 1, _BLK), lambda i: (i, 0, 0))],
        out_shape=[jax.ShapeDtypeStruct((_GRID, 1, _BLK), jnp.float32),
                   jax.ShapeDtypeStruct((_GRID, 1, _BLK), jnp.float32)],
    )(x2, degp)


def _tc_c1(p0p_ref, xt_ref, dis_ref, w1_ref, b1_ref, o_ht1):
    q = (p0p_ref[0] + p0p_ref[1] + xt_ref[0, 0]) * dis_ref[0, 0]
    h1 = jnp.maximum(q[:, None] * w1_ref[0][None, :] + b1_ref[0][None, :], 0.0)
    ht1 = dis_ref[0, 0][:, None] * h1
    o_ht1[0] = ht1[:, :32]
    o_ht1[1] = ht1[:, 32:]


def _c1_call(p0p, xt2, dis2, W1, b1):
    return pl.pallas_call(
        _tc_c1,
        grid=(_GRID,),
        in_specs=[
            pl.BlockSpec((2, _BLK), lambda i: (0, i)),
            pl.BlockSpec((1, 1, _BLK), lambda i: (i, 0, 0)),
            pl.BlockSpec((1, 1, _BLK), lambda i: (i, 0, 0)),
            pl.BlockSpec((1, 64), lambda i: (0, 0)),
            pl.BlockSpec((1, 64), lambda i: (0, 0)),
        ],
        out_specs=pl.BlockSpec((2, _BLK, 32), lambda i: (0, i, 0)),
        out_shape=jax.ShapeDtypeStruct((2, NROWS, 32), jnp.float32),
    )(p0p, xt2, dis2, W1, b1.reshape(1, 64))


def _tc_e(s1_ref, ht1_ref, dis_ref, w2_ref, b2_ref, o_ht2):
    dis = dis_ref[0, 0]
    q = jnp.concatenate([s1_ref[0] + ht1_ref[0], s1_ref[1] + ht1_ref[1]],
                        axis=1) * dis[:, None]
    h2 = jnp.dot(q, w2_ref[...], preferred_element_type=jnp.float32)
    h2 = jnp.maximum(h2 + b2_ref[0][None, :], 0.0)
    ht2 = dis[:, None] * h2
    for j in range(4):
        o_ht2[j] = ht2[:, j * 32:(j + 1) * 32]


def _e_call(s1, ht1, dis2, W2, b2):
    return pl.pallas_call(
        _tc_e,
        grid=(_GRID,),
        in_specs=[
            pl.BlockSpec((2, _BLK, 32), lambda i: (0, i, 0)),
            pl.BlockSpec((2, _BLK, 32), lambda i: (0, i, 0)),
            pl.BlockSpec((1, 1, _BLK), lambda i: (i, 0, 0)),
            pl.BlockSpec((64, 128), lambda i: (0, 0)),
            pl.BlockSpec((1, 128), lambda i: (0, 0)),
        ],
        out_specs=pl.BlockSpec((4, _BLK, 32), lambda i: (0, i, 0)),
        out_shape=jax.ShapeDtypeStruct((4, NROWS, 32), jnp.float32),
    )(s1, ht1, dis2, W2, b2.reshape(1, 128))


def _tc_g(s2_ref, ht2_ref, dis_ref, w3_ref, b3_ref, o_h3):
    dis = dis_ref[0, 0]
    q = jnp.concatenate([s2_ref[j] + ht2_ref[j] for j in range(4)],
                        axis=1) * dis[:, None]
    h3 = jnp.dot(q, w3_ref[...], preferred_element_type=jnp.float32)
    o_h3[...] = h3 + b3_ref[0][None, :]


def _g_call(s2, ht2, dis2, W3, b3):
    return pl.pallas_call(
        _tc_g,
        grid=(_GRID,),
        in_specs=[
            pl.BlockSpec((4, _BLK, 32), lambda i: (0, i, 0)),
            pl.BlockSpec((4, _BLK, 32), lambda i: (0, i, 0)),
            pl.BlockSpec((1, 1, _BLK), lambda i: (i, 0, 0)),
            pl.BlockSpec((128, 256), lambda i: (0, 0)),
            pl.BlockSpec((1, 256), lambda i: (0, 0)),
        ],
        out_specs=pl.BlockSpec((_BLK, 256), lambda i: (i, 0)),
        out_shape=jax.ShapeDtypeStruct((NROWS, 256), jnp.float32),
    )(s2, ht2, dis2, W3, b3.reshape(1, 256))


def _tc_head(psum_ref, pmax_ref, cntp_ref, fw1_ref, fb1_ref, fw2_ref, fb2_ref,
             o_ref):
    counts = cntp_ref[0, :G] + cntp_ref[1, :G]
    mean = psum_ref[:G] / jnp.maximum(counts, 1.0)[:, None]
    mx = jnp.where(counts[:, None] > 0, pmax_ref[:G], 0.0)
    z = jnp.concatenate([mean, mx], axis=1)
    z = jnp.dot(z, fw1_ref[...], preferred_element_type=jnp.float32)
    z = jnp.maximum(z + fb1_ref[0][None, :], 0.0)
    out = jnp.dot(z, fw2_ref[...], preferred_element_type=jnp.float32)
    o_ref[...] = out + fb2_ref[0][None, :]


def _head_call(psum, pmax, cntp, fW1, fb1, fW2, fb2):
    return pl.pallas_call(
        _tc_head,
        grid=(1,),
        in_specs=[
            pl.BlockSpec((GPAD, 256), lambda i: (0, 0)),
            pl.BlockSpec((GPAD, 256), lambda i: (0, 0)),
            pl.BlockSpec((2, GPAD), lambda i: (0, 0)),
            pl.BlockSpec((512, 128), lambda i: (0, 0)),
            pl.BlockSpec((1, 128), lambda i: (0, 0)),
            pl.BlockSpec((128, 12), lambda i: (0, 0)),
            pl.BlockSpec((1, 12), lambda i: (0, 0)),
        ],
        out_specs=pl.BlockSpec((G, 12), lambda i: (0, 0)),
        out_shape=jax.ShapeDtypeStruct((G, 12), jnp.float32),
    )(psum, pmax, cntp, fW1, fb1.reshape(1, 128), fW2, fb2.reshape(1, 12))


def kernel(x, edge_index, batch, W1, b1, W2, b2, W3, b3, fW1, fb1, fW2, fb2):
    src = edge_index[0]
    dst = edge_index[1]
    bat_pad = jnp.concatenate(
        [batch, jnp.full((NROWS - N,), G, jnp.int32)])
    x2 = jnp.pad(x[:, 0], (0, NROWS - N)).reshape(_GRID, 1, _BLK)

    src2 = src.reshape(ECH, 128)
    dst2 = dst.reshape(ECH, 128)
    degp, cntp = _deg_counts(dst2, bat_pad.reshape(BCH, 128))
    dis2, xt2 = _c0_call(x2, degp)
    (p0p,) = _p0_scatter(xt2.reshape(NROWS), src2, dst2)
    ht1 = _c1_call(p0p, xt2, dis2, W1, b1)
    (s1,) = _s1_scatter(ht1, src2, dst2)
    ht2 = _e_call(s1, ht1, dis2, W2, b2)
    (s2,) = _s2_scatter(ht2, src2, dst2)
    h3 = _g_call(s2, ht2, dis2, W3, b3)
    psum, pmax = _pool(h3.reshape(NROWS * 256), cntp)
    return _head_call(psum, pmax, cntp, fW1, fb1, fW2, fb2)


# speedup vs baseline: 32.0310x; 1.0202x over previous
import functools
import jax
import jax.numpy as jnp
from jax import lax
from jax.experimental import pallas as pl
from jax.experimental.pallas import tpu as pltpu
from jax.experimental.pallas import tpu_sc as plsc

N = 50000
E = 800000
G = 2000
NPAD = 51200
NROWS = 50176
GPAD = 2048
ECH = E // 128
BCH = NROWS // 128
NEG_INF = float("-inf")

_MESH = plsc.VectorSubcoreMesh(core_axis_name="c", subcore_axis_name="s")
_Z16 = functools.partial(jnp.zeros, (16,), jnp.float32)


def _fill(ref, n, value):
    v = jnp.full((16,), value, jnp.float32)

    def body(i, _):
        ref[pl.ds(i * 16, 16)] = v
        return 0

    lax.fori_loop(0, n // 16, body, 0)


def _fill2d(ref, rows, value):
    v = jnp.full((16,), value, jnp.float32)

    def body(i, _):
        r = i // 2
        col = (i % 2) * 16
        ref[r, pl.ds(col, 16)] = v
        return 0

    lax.fori_loop(0, rows * 2, body, 0)


def _sc_deg_counts(dst2_hbm, bat2_hbm, degp_hbm, cntp_hbm,
                   accd, accc, zbuf, obuf, idxd0, idxd1, idxd2,
                   ss0, ss1, ss2):
    c = lax.axis_index("c")
    s = lax.axis_index("s")
    w = c * 16 + s
    KB = 8
    NSB = 8
    MAIN = 32 * 192
    TAIL = ECH - MAIN
    _fill(zbuf, 3200, 0.0)
    _fill(obuf, 1024, 1.0)
    pltpu.sync_copy(zbuf, accd.at[pl.ds(s * 3200, 3200)])
    pltpu.sync_copy(zbuf.at[pl.ds(0, 128)], accc.at[pl.ds(s * 128, 128)])
    plsc.subcore_barrier()
    idxd = (idxd0, idxd1, idxd2)
    ss = (ss0, ss1, ss2)

    def drain_scatters(q):
        for b in range(KB):
            pltpu.make_async_copy(obuf.at[pl.ds(b * 128, 128)],
                                  accd.at[idxd[q].at[b]], ss[q]).wait()

    def do_block(sb, q):
        row0 = w * 192 + (3 * sb + q) * KB

        @pl.when(sb > 0)
        def _():
            drain_scatters(q)

        pltpu.sync_copy(dst2_hbm.at[pl.ds(row0, KB), :], idxd[q])
        for b in range(KB):
            pltpu.async_copy(obuf.at[pl.ds(b * 128, 128)],
                             accd.at[idxd[q].at[b]], ss[q], add=True)

    def sbstep(sb, _):
        do_block(sb, 0)
        do_block(sb, 1)
        do_block(sb, 2)
        return 0

    lax.fori_loop(0, NSB, sbstep, 0)
    drain_scatters(0)
    drain_scatters(1)
    drain_scatters(2)

    def tstep(k, _):
        pltpu.sync_copy(dst2_hbm.at[pl.ds(k, 1), :], idxd0.at[pl.ds(0, 1), :])
        pltpu.sync_copy(obuf.at[pl.ds(0, 128)], accd.at[idxd0.at[0]], add=True)
        return 0

    lax.fori_loop(MAIN + w * TAIL // 32, MAIN + (w + 1) * TAIL // 32, tstep, 0)

    def bstep(k, _):
        pltpu.sync_copy(bat2_hbm.at[pl.ds(k, 1), :], idxd0.at[pl.ds(0, 1), :])
        pltpu.sync_copy(obuf.at[pl.ds(0, 128)], accc.at[idxd0.at[0]], add=True)
        return 0

    lax.fori_loop(w * BCH // 32, (w + 1) * BCH // 32, bstep, 0)

    plsc.subcore_barrier()
    pltpu.sync_copy(accd.at[pl.ds(s * 3200, 3200)],
                    degp_hbm.at[c].at[pl.ds(s * 3200, 3200)])
    pltpu.sync_copy(accc.at[pl.ds(s * 128, 128)],
                    cntp_hbm.at[c].at[pl.ds(s * 128, 128)])


_deg_counts = pl.kernel(
    _sc_deg_counts,
    out_type=[jax.ShapeDtypeStruct((2, NPAD), jnp.float32),
              jax.ShapeDtypeStruct((2, GPAD), jnp.float32)],
    mesh=_MESH,
    compiler_params=pltpu.CompilerParams(use_tc_tiling_on_sc=False),
    scratch_types=[
        pltpu.VMEM_SHARED((NPAD,), jnp.float32),
        pltpu.VMEM_SHARED((GPAD,), jnp.float32),
        pltpu.VMEM((3200,), jnp.float32),
        pltpu.VMEM((1024,), jnp.float32),
        pltpu.VMEM((8, 128), jnp.int32),
        pltpu.VMEM((8, 128), jnp.int32),
        pltpu.VMEM((8, 128), jnp.int32),
        pltpu.SemaphoreType.DMA,
        pltpu.SemaphoreType.DMA,
        pltpu.SemaphoreType.DMA,
    ],
)


def _sc_p0(xt_hbm, src2_hbm, dst2_hbm, p0p_hbm,
           acc, zbuf, vals0, vals1, vals2,
           idxs0, idxs1, idxs2, idxd0, idxd1, idxd2,
           sg0, sg1, sg2, ss0, ss1, ss2, st):
    c = lax.axis_index("c")
    s = lax.axis_index("s")
    w = c * 16 + s
    KB = 8
    NSB = 8
    MAIN = 32 * 192
    TAIL = ECH - MAIN
    _fill(zbuf, 3200, 0.0)
    pltpu.sync_copy(zbuf, acc.at[pl.ds(s * 3200, 3200)])
    plsc.subcore_barrier()
    vals = (vals0, vals1, vals2)
    idxs = (idxs0, idxs1, idxs2)
    idxd = (idxd0, idxd1, idxd2)
    sg = (sg0, sg1, sg2)
    ss = (ss0, ss1, ss2)

    def drain_scatters(q):
        for b in range(KB):
            pltpu.make_async_copy(vals[q].at[pl.ds(b * 128, 128)],
                                  acc.at[idxd[q].at[b]], ss[q]).wait()

    def drain_gathers(q):
        for b in range(KB):
            pltpu.make_async_copy(xt_hbm.at[idxs[q].at[b]],
                                  vals[q].at[pl.ds(b * 128, 128)], sg[q]).wait()

    def fire_scatters(q):
        for b in range(KB):
            pltpu.async_copy(vals[q].at[pl.ds(b * 128, 128)],
                             acc.at[idxd[q].at[b]], ss[q], add=True)

    def do_block(sb, q):
        qp = (q - 1) % 3
        row0 = w * 192 + (3 * sb + q) * KB
        @pl.when(sb > 0)
        def _():
            drain_scatters(q)
        pltpu.sync_copy(src2_hbm.at[pl.ds(row0, KB), :], idxs[q])
        pltpu.sync_copy(dst2_hbm.at[pl.ds(row0, KB), :], idxd[q])
        for b in range(KB):
            pltpu.async_copy(xt_hbm.at[idxs[q].at[b]],
                             vals[q].at[pl.ds(b * 128, 128)], sg[q])
        if q == 0:
            @pl.when(sb > 0)
            def _():
                drain_gathers(qp)
                fire_scatters(qp)
        else:
            drain_gathers(qp)
            fire_scatters(qp)

    def sbstep(sb, _):
        do_block(sb, 0)
        do_block(sb, 1)
        do_block(sb, 2)
        return 0

    lax.fori_loop(0, NSB, sbstep, 0)
    drain_gathers(2)
    fire_scatters(2)
    drain_scatters(0)
    drain_scatters(1)
    drain_scatters(2)

    def tstep(k, _):
        pltpu.sync_copy(src2_hbm.at[pl.ds(k, 1), :], idxs0.at[pl.ds(0, 1), :])
        pltpu.async_copy(xt_hbm.at[idxs0.at[0]],
                         vals0.at[pl.ds(0, 128)], st).wait()
        pltpu.sync_copy(dst2_hbm.at[pl.ds(k, 1), :], idxd0.at[pl.ds(0, 1), :])
        pltpu.sync_copy(vals0.at[pl.ds(0, 128)],
                        acc.at[idxd0.at[0]], add=True)
        return 0

    lax.fori_loop(MAIN + w * TAIL // 32, MAIN + (w + 1) * TAIL // 32, tstep, 0)

    plsc.subcore_barrier()
    pltpu.sync_copy(acc.at[pl.ds(s * 3200, 3200)],
                    p0p_hbm.at[c].at[pl.ds(s * 3200, 3200)])


_p0_scatter = pl.kernel(
    _sc_p0,
    out_type=[jax.ShapeDtypeStruct((2, NPAD), jnp.float32)],
    mesh=_MESH,
    compiler_params=pltpu.CompilerParams(use_tc_tiling_on_sc=False),
    scratch_types=[
        pltpu.VMEM_SHARED((NPAD,), jnp.float32),
        pltpu.VMEM((3200,), jnp.float32),
        pltpu.VMEM((1024,), jnp.float32),
        pltpu.VMEM((1024,), jnp.float32),
        pltpu.VMEM((1024,), jnp.float32),
        pltpu.VMEM((8, 128), jnp.int32),
        pltpu.VMEM((8, 128), jnp.int32),
        pltpu.VMEM((8, 128), jnp.int32),
        pltpu.VMEM((8, 128), jnp.int32),
        pltpu.VMEM((8, 128), jnp.int32),
        pltpu.VMEM((8, 128), jnp.int32),
        pltpu.SemaphoreType.DMA,
        pltpu.SemaphoreType.DMA,
        pltpu.SemaphoreType.DMA,
        pltpu.SemaphoreType.DMA,
        pltpu.SemaphoreType.DMA,
        pltpu.SemaphoreType.DMA,
        pltpu.SemaphoreType.DMA,
    ],
)


NPAD2 = 50048


def _make_row_scatter(num_chunks):
    per_core = num_chunks // 2
    KB = 2
    NSB = 32
    MAIN = 16 * 384
    TAIL = ECH - MAIN

    def body(ht_hbm, src2_hbm, dst2_hbm, out_hbm,
             acc, zrows, rows0, rows1, rows2,
             idxs0, idxs1, idxs2, idxs3, idxs4, idxs5,
             idxd0, idxd1, idxd2, idxd3, idxd4, idxd5,
             sg0, sg1, sg2, ss0, ss1, ss2,
             si0, si1, si2, si3, si4, si5, st):
        c = lax.axis_index("c")
        s = lax.axis_index("s")
        _fill2d(zrows, 32, 0.0)
        rows = (rows0, rows1, rows2)
        idxs = (idxs0, idxs1, idxs2, idxs3, idxs4, idxs5)
        idxd = (idxd0, idxd1, idxd2, idxd3, idxd4, idxd5)
        sg = (sg0, sg1, sg2)
        ss = (ss0, ss1, ss2)
        si = (si0, si1, si2, si3, si4, si5)

        def row0_of(sb, q):
            return s * 384 + (6 * sb + q) * KB

        def drain_scatters(r, qi):
            for b in range(KB):
                pltpu.make_async_copy(
                    rows[r].at[pl.ds(b * 128, 128), :],
                    acc.at[idxd[qi].at[b]], ss[r]).wait()

        def drain_gathers(r, qi):
            for b in range(KB):
                pltpu.make_async_copy(
                    ht_hbm.at[0].at[idxs[qi].at[b]],
                    rows[r].at[pl.ds(b * 128, 128), :], sg[r]).wait()

        def fire_scatters(r, qi):
            for b in range(KB):
                pltpu.async_copy(rows[r].at[pl.ds(b * 128, 128), :],
                                 acc.at[idxd[qi].at[b]], ss[r], add=True)

        def issue_idx(sb, q):
            r0 = row0_of(sb, q) if q < 6 else row0_of(sb + 1, q - 6)
            qi = q % 6
            pltpu.async_copy(src2_hbm.at[pl.ds(r0, KB), :], idxs[qi], si[qi])
            pltpu.async_copy(dst2_hbm.at[pl.ds(r0, KB), :], idxd[qi], si[qi])

        def wait_idx(qi):
            pltpu.make_async_copy(src2_hbm.at[pl.ds(0, KB), :],
                                  idxs[qi], si[qi]).wait()
            pltpu.make_async_copy(dst2_hbm.at[pl.ds(0, KB), :],
                                  idxd[qi], si[qi]).wait()

        for j in range(per_core):
            ch = c * per_core + j

            def zstep(i, _):
                pltpu.async_copy(zrows, acc.at[pl.ds(s * 3128 + i * 32, 32), :],
                                 st)
                return 0

            lax.fori_loop(0, 97, zstep, 0)
            pltpu.sync_copy(zrows.at[pl.ds(0, 24), :],
                            acc.at[pl.ds(s * 3128 + 3104, 24), :])

            def zwait(i, _):
                pltpu.make_async_copy(
                    zrows, acc.at[pl.ds(s * 3128, 32), :], st).wait()
                return 0

            lax.fori_loop(0, 97, zwait, 0)
            plsc.subcore_barrier()

            def fire_gathers(r, qi):
                for b in range(KB):
                    pltpu.async_copy(ht_hbm.at[ch].at[idxs[qi].at[b]],
                                     rows[r].at[pl.ds(b * 128, 128), :], sg[r])

            def do_block(sb, q):
                r = q % 3
                rp = (q - 1) % 3
                qp = (q - 1) % 6

                if q < 2:
                    @pl.when(sb > 0)
                    def _():
                        wait_idx(q)
                else:
                    wait_idx(q)

                if q < 3:
                    @pl.when(sb > 0)
                    def _():
                        drain_scatters(r, (q + 3) % 6)
                else:
                    drain_scatters(r, (q + 3) % 6)

                fire_gathers(r, q)

                if q < 4:
                    issue_idx(sb, q + 2)
                else:
                    @pl.when(sb < NSB - 1)
                    def _():
                        issue_idx(sb, q + 2)

                if q == 0:
                    @pl.when(sb > 0)
                    def _():
                        drain_gathers(rp, 5)
                        fire_scatters(rp, 5)
                else:
                    drain_gathers(rp, qp)
                    fire_scatters(rp, qp)

            def sbstep(sb, _):
                for q in range(6):
                    do_block(sb, q)
                return 0

            pltpu.sync_copy(src2_hbm.at[pl.ds(row0_of(0, 0), KB), :], idxs0)
            pltpu.sync_copy(dst2_hbm.at[pl.ds(row0_of(0, 0), KB), :], idxd0)
            pltpu.sync_copy(src2_hbm.at[pl.ds(row0_of(0, 1), KB), :], idxs1)
            pltpu.sync_copy(dst2_hbm.at[pl.ds(row0_of(0, 1), KB), :], idxd1)
            lax.fori_loop(0, NSB, sbstep, 0)
            drain_gathers(2, 5)
            fire_scatters(2, 5)
            drain_scatters(0, 3)
            drain_scatters(1, 4)
            drain_scatters(2, 5)

            def tstep(k, _):
                pltpu.sync_copy(src2_hbm.at[pl.ds(k, 1), :],
                                idxs0.at[pl.ds(0, 1), :])
                pltpu.async_copy(ht_hbm.at[ch].at[idxs0.at[0]],
                                 rows0.at[pl.ds(0, 128), :], st).wait()
                pltpu.sync_copy(dst2_hbm.at[pl.ds(k, 1), :],
                                idxd0.at[pl.ds(0, 1), :])
                pltpu.sync_copy(rows0.at[pl.ds(0, 128), :],
                                acc.at[idxd0.at[0]], add=True)
                return 0

            lax.fori_loop(MAIN + s * TAIL // 16, MAIN + (s + 1) * TAIL // 16,
                          tstep, 0)

            plsc.subcore_barrier()
            pltpu.sync_copy(acc.at[pl.ds(s * 3128, 3128), :],
                            out_hbm.at[ch].at[pl.ds(s * 3128, 3128), :])
            plsc.subcore_barrier()

    return pl.kernel(
        body,
        out_type=[jax.ShapeDtypeStruct((num_chunks, NPAD2, 32), jnp.float32)],
        mesh=_MESH,
        compiler_params=pltpu.CompilerParams(use_tc_tiling_on_sc=False),
        scratch_types=[
            pltpu.VMEM_SHARED((NPAD2, 32), jnp.float32),
            pltpu.VMEM((32, 32), jnp.float32),
            pltpu.VMEM((256, 32), jnp.float32),
            pltpu.VMEM((256, 32), jnp.float32),
            pltpu.VMEM((256, 32), jnp.float32),
        ] + [pltpu.VMEM((2, 128), jnp.int32)] * 12
          + [pltpu.SemaphoreType.DMA] * 13,
    )


_s1_scatter = _make_row_scatter(2)
_s2_scatter = _make_row_scatter(4)


def _sc_pool(h3_hbm, cntp_hbm, psum_hbm, pmax_hbm,
             cbuf, cbuf2, rbuf0, rbuf1, osum, omax, sp0, sp1):
    c = lax.axis_index("c")
    s = lax.axis_index("s")
    wid = c * 16 + s
    pltpu.sync_copy(cntp_hbm.at[0], cbuf.at[pl.ds(0, GPAD)])
    pltpu.sync_copy(cntp_hbm.at[1], cbuf2)

    def addc(i, _):
        sl = pl.ds(i * 16, 16)
        cbuf[sl] = cbuf[sl] + cbuf2[sl]
        return 0

    lax.fori_loop(0, GPAD // 16, addc, 0)

    def pre(i, acc16):
        return acc16 + cbuf[pl.ds(i * 16, 16)]

    acc16 = lax.fori_loop(0, wid * 4, pre, _Z16())
    start0 = jnp.sum(acc16).astype(jnp.int32)
    rbuf = (rbuf0, rbuf1)
    sp = (sp0, sp1)

    def clamp(p):
        return jnp.minimum(p, NROWS - 32)

    pltpu.async_copy(h3_hbm.at[pl.ds(clamp(start0) * 256, 32 * 256)],
                     rbuf0, sp0)

    def one_graph(g_local, start, sl):
        g = wid * 64 + g_local
        cnt = cbuf[pl.ds(g, 16)][0].astype(jnp.int32)
        nch = (cnt + 31) // 32
        pltpu.make_async_copy(
            h3_hbm.at[pl.ds(0, 32 * 256)], rbuf[sl], sp[sl]).wait()
        pltpu.async_copy(
            h3_hbm.at[pl.ds(clamp(start + cnt) * 256, 32 * 256)],
            rbuf[1 - sl], sp[1 - sl])

        def rows_of(carry, lo, hi):
            def row_step(r, carry2):
                sums, maxs = carry2
                new_s = []
                new_m = []
                for jj in range(16):
                    v = rbuf[sl][pl.ds(r * 256 + jj * 16, 16)]
                    new_s.append(sums[jj] + v)
                    new_m.append(jnp.maximum(maxs[jj], v))
                return tuple(new_s), tuple(new_m)

            return lax.fori_loop(lo, hi, row_step, carry)

        init = (tuple(_Z16() for _ in range(16)),
                tuple(jnp.full((16,), NEG_INF, jnp.float32) for _ in range(16)))
        carry = rows_of(init, 0, jnp.minimum(32, cnt))

        def chunk_step(k, carry):
            pltpu.sync_copy(
                h3_hbm.at[pl.ds(clamp(start + k * 32) * 256, 32 * 256)],
                rbuf[sl])
            return rows_of(carry, 0, jnp.minimum(32, cnt - k * 32))

        sums, maxs = lax.fori_loop(1, nch, chunk_step, carry)
        for jj in range(16):
            osum[g_local, pl.ds(jj * 16, 16)] = sums[jj]
            omax[g_local, pl.ds(jj * 16, 16)] = maxs[jj]
        return start + cnt

    def pair_step(i, start):
        start = one_graph(2 * i, start, 0)
        start = one_graph(2 * i + 1, start, 1)
        return start

    lax.fori_loop(0, 32, pair_step, start0)
    pltpu.make_async_copy(h3_hbm.at[pl.ds(0, 32 * 256)], rbuf0, sp0).wait()
    pltpu.sync_copy(osum, psum_hbm.at[pl.ds(wid * 64, 64), :])
    pltpu.sync_copy(omax, pmax_hbm.at[pl.ds(wid * 64, 64), :])


_pool = pl.kernel(
    _sc_pool,
    out_type=[jax.ShapeDtypeStruct((GPAD, 256), jnp.float32),
              jax.ShapeDtypeStruct((GPAD, 256), jnp.float32)],
    mesh=_MESH,
    compiler_params=pltpu.CompilerParams(needs_layout_passes=False),
    scratch_types=[
        pltpu.VMEM((GPAD + 16,), jnp.float32),
        pltpu.VMEM((GPAD,), jnp.float32),
        pltpu.VMEM((32 * 256,), jnp.float32),
        pltpu.VMEM((32 * 256,), jnp.float32),
        pltpu.VMEM((64, 256), jnp.float32),
        pltpu.VMEM((64, 256), jnp.float32),
        pltpu.SemaphoreType.DMA,
        pltpu.SemaphoreType.DMA,
    ],
)


_BLK = 1024
_GRID = NROWS // _BLK


def _tc_c0(x_ref, degp_ref, o_dis, o_xt):
    deg = degp_ref[0] + degp_ref[1] + 1.0
    dis = lax.rsqrt(deg)
    o_dis[0, 0] = dis
    o_xt[0, 0] = dis * x_ref[0, 0]


def _c0_call(x2, degp):
    return pl.pallas_call(
        _tc_c0,
        grid=(_GRID,),
        in_specs=[
            pl.BlockSpec((1, 1, _BLK), lambda i: (i, 0, 0)),
            pl.BlockSpec((2, _BLK), lambda i: (0, i)),
        ],
        out_specs=[pl.BlockSpec((1, 1, _BLK), lambda i: (i, 0, 0)),
                   pl.BlockSpec((1, 1, _BLK), lambda i: (i, 0, 0))],
        out_shape=[jax.ShapeDtypeStruct((_GRID, 1, _BLK), jnp.float32),
                   jax.ShapeDtypeStruct((_GRID, 1, _BLK), jnp.float32)],
    )(x2, degp)


def _tc_c1(p0p_ref, xt_ref, dis_ref, w1_ref, b1_ref, o_ht1):
    q = (p0p_ref[0] + p0p_ref[1] + xt_ref[0, 0]) * dis_ref[0, 0]
    h1 = jnp.maximum(q[:, None] * w1_ref[0][None, :] + b1_ref[0][None, :], 0.0)
    ht1 = dis_ref[0, 0][:, None] * h1
    o_ht1[0] = ht1[:, :32]
    o_ht1[1] = ht1[:, 32:]


def _c1_call(p0p, xt2, dis2, W1, b1):
    return pl.pallas_call(
        _tc_c1,
        grid=(_GRID,),
        in_specs=[
            pl.BlockSpec((2, _BLK), lambda i: (0, i)),
            pl.BlockSpec((1, 1, _BLK), lambda i: (i, 0, 0)),
            pl.BlockSpec((1, 1, _BLK), lambda i: (i, 0, 0)),
            pl.BlockSpec((1, 64), lambda i: (0, 0)),
            pl.BlockSpec((1, 64), lambda i: (0, 0)),
        ],
        out_specs=pl.BlockSpec((2, _BLK, 32), lambda i: (0, i, 0)),
        out_shape=jax.ShapeDtypeStruct((2, NROWS, 32), jnp.float32),
    )(p0p, xt2, dis2, W1, b1.reshape(1, 64))


def _tc_e(s1_ref, ht1_ref, dis_ref, w2_ref, b2_ref, o_ht2):
    dis = dis_ref[0, 0]
    q = jnp.concatenate([s1_ref[0] + ht1_ref[0], s1_ref[1] + ht1_ref[1]],
                        axis=1) * dis[:, None]
    h2 = jnp.dot(q, w2_ref[...], preferred_element_type=jnp.float32)
    h2 = jnp.maximum(h2 + b2_ref[0][None, :], 0.0)
    ht2 = dis[:, None] * h2
    for j in range(4):
        o_ht2[j] = ht2[:, j * 32:(j + 1) * 32]


def _e_call(s1, ht1, dis2, W2, b2):
    return pl.pallas_call(
        _tc_e,
        grid=(_GRID,),
        in_specs=[
            pl.BlockSpec((2, _BLK, 32), lambda i: (0, i, 0)),
            pl.BlockSpec((2, _BLK, 32), lambda i: (0, i, 0)),
            pl.BlockSpec((1, 1, _BLK), lambda i: (i, 0, 0)),
            pl.BlockSpec((64, 128), lambda i: (0, 0)),
            pl.BlockSpec((1, 128), lambda i: (0, 0)),
        ],
        out_specs=pl.BlockSpec((4, _BLK, 32), lambda i: (0, i, 0)),
        out_shape=jax.ShapeDtypeStruct((4, NROWS, 32), jnp.float32),
    )(s1, ht1, dis2, W2, b2.reshape(1, 128))


def _tc_g(s2_ref, ht2_ref, dis_ref, w3_ref, b3_ref, o_h3):
    dis = dis_ref[0, 0]
    q = jnp.concatenate([s2_ref[j] + ht2_ref[j] for j in range(4)],
                        axis=1) * dis[:, None]
    h3 = jnp.dot(q, w3_ref[...], preferred_element_type=jnp.float32)
    o_h3[...] = h3 + b3_ref[0][None, :]


def _g_call(s2, ht2, dis2, W3, b3):
    return pl.pallas_call(
        _tc_g,
        grid=(_GRID,),
        in_specs=[
            pl.BlockSpec((4, _BLK, 32), lambda i: (0, i, 0)),
            pl.BlockSpec((4, _BLK, 32), lambda i: (0, i, 0)),
            pl.BlockSpec((1, 1, _BLK), lambda i: (i, 0, 0)),
            pl.BlockSpec((128, 256), lambda i: (0, 0)),
            pl.BlockSpec((1, 256), lambda i: (0, 0)),
        ],
        out_specs=pl.BlockSpec((_BLK, 256), lambda i: (i, 0)),
        out_shape=jax.ShapeDtypeStruct((NROWS, 256), jnp.float32),
    )(s2, ht2, dis2, W3, b3.reshape(1, 256))


def _tc_head(psum_ref, pmax_ref, cntp_ref, fw1_ref, fb1_ref, fw2_ref, fb2_ref,
             o_ref):
    counts = cntp_ref[0, :G] + cntp_ref[1, :G]
    mean = psum_ref[:G] / jnp.maximum(counts, 1.0)[:, None]
    mx = jnp.where(counts[:, None] > 0, pmax_ref[:G], 0.0)
    z = jnp.concatenate([mean, mx], axis=1)
    z = jnp.dot(z, fw1_ref[...], preferred_element_type=jnp.float32)
    z = jnp.maximum(z + fb1_ref[0][None, :], 0.0)
    out = jnp.dot(z, fw2_ref[...], preferred_element_type=jnp.float32)
    o_ref[...] = out + fb2_ref[0][None, :]


def _head_call(psum, pmax, cntp, fW1, fb1, fW2, fb2):
    return pl.pallas_call(
        _tc_head,
        grid=(1,),
        in_specs=[
            pl.BlockSpec((GPAD, 256), lambda i: (0, 0)),
            pl.BlockSpec((GPAD, 256), lambda i: (0, 0)),
            pl.BlockSpec((2, GPAD), lambda i: (0, 0)),
            pl.BlockSpec((512, 128), lambda i: (0, 0)),
            pl.BlockSpec((1, 128), lambda i: (0, 0)),
            pl.BlockSpec((128, 12), lambda i: (0, 0)),
            pl.BlockSpec((1, 12), lambda i: (0, 0)),
        ],
        out_specs=pl.BlockSpec((G, 12), lambda i: (0, 0)),
        out_shape=jax.ShapeDtypeStruct((G, 12), jnp.float32),
    )(psum, pmax, cntp, fW1, fb1.reshape(1, 128), fW2, fb2.reshape(1, 12))


def kernel(x, edge_index, batch, W1, b1, W2, b2, W3, b3, fW1, fb1, fW2, fb2):
    src = edge_index[0]
    dst = edge_index[1]
    bat_pad = jnp.concatenate(
        [batch, jnp.full((NROWS - N,), G, jnp.int32)])
    x2 = jnp.pad(x[:, 0], (0, NROWS - N)).reshape(_GRID, 1, _BLK)

    src2 = src.reshape(ECH, 128)
    dst2 = dst.reshape(ECH, 128)
    degp, cntp = _deg_counts(dst2, bat_pad.reshape(BCH, 128))
    dis2, xt2 = _c0_call(x2, degp)
    (p0p,) = _p0_scatter(xt2.reshape(NROWS), src2, dst2)
    ht1 = _c1_call(p0p, xt2, dis2, W1, b1)
    (s1,) = _s1_scatter(ht1, src2, dst2)
    ht2 = _e_call(s1, ht1, dis2, W2, b2)
    (s2,) = _s2_scatter(ht2, src2, dst2)
    h3 = _g_call(s2, ht2, dis2, W3, b3)
    psum, pmax = _pool(h3.reshape(NROWS * 256), cntp)
    return _head_call(psum, pmax, cntp, fW1, fb1, fW2, fb2)
